# Initial kernel scaffold; baseline (speedup 1.0000x reference)
import jax
import jax.numpy as jnp
from jax.experimental import pallas as pl


def kernel(api_embeds, adjacency_matrix, edge_index, Wr1, br1, Wr2, br2, m1W1, m1b1, m1W2, m1b2, m2W1, m2b1, m2W2, m2b2, moW1, mob1, moW2, mob2, Wi1, bi1, Wi2, bi2):
    raise NotImplementedError("write your pallas kernel here")


# speedup vs baseline: 2.6123x; 2.6123x over previous
import functools

import jax
import jax.numpy as jnp
from jax import lax
from jax.experimental import pallas as pl
from jax.experimental.pallas import tpu as pltpu
from jax.experimental.pallas import tpu_sc as plsc

D = 128
N = 10000
NUM_API = 10000
E = 160000

NC = 2
NS = 16
C = 128
NCHUNK = E // C


def _elu(x):
    return jnp.where(x > 0, x, jnp.exp(jnp.minimum(x, 0.0)) - 1.0)


def _gather_body(a_hbm, dst_hbm, src_hbm, g1_hbm, g2_hbm,
                 idx1_v, idx2_v, buf1_v, buf2_v, sem1, sem2):
    wid = lax.axis_index("s") * NC + lax.axis_index("c")
    per = (NCHUNK + NC * NS - 1) // (NC * NS)

    def chunk_body(i, _):
        chunk = wid * per + i

        @pl.when(chunk < NCHUNK)
        def _():
            base = chunk * C
            pltpu.sync_copy(dst_hbm.at[pl.ds(base, C)], idx1_v)
            pltpu.sync_copy(src_hbm.at[pl.ds(base, C)], idx2_v)
            cp1 = pltpu.async_copy(a_hbm.at[idx1_v], buf1_v, sem1)
            cp2 = pltpu.async_copy(a_hbm.at[idx2_v], buf2_v, sem2)
            cp1.wait()
            cp2.wait()
            pltpu.sync_copy(buf1_v, g1_hbm.at[pl.ds(base, C)])
            pltpu.sync_copy(buf2_v, g2_hbm.at[pl.ds(base, C)])
        return 0

    lax.fori_loop(0, per, chunk_body, 0)


_sc_gather = functools.partial(
    pl.kernel,
    out_type=(jax.ShapeDtypeStruct((E, D), jnp.float32),
              jax.ShapeDtypeStruct((E, D), jnp.float32)),
    mesh=plsc.VectorSubcoreMesh(core_axis_name="c", subcore_axis_name="s",
                                num_cores=NC, num_subcores=NS),
    scratch_types=(pltpu.VMEM((C,), jnp.int32),
                   pltpu.VMEM((C,), jnp.int32),
                   pltpu.VMEM((C, D), jnp.float32),
                   pltpu.VMEM((C, D), jnp.float32),
                   pltpu.SemaphoreType.DMA,
                   pltpu.SemaphoreType.DMA),
)(_gather_body)


ROWS_PER_TILE = 1000


def _scatter_body(msg_hbm, dst_hbm, zeros_hbm, agg_hbm,
                  accum_sh, idx_v, buf_v, sem):
    c = lax.axis_index("c")
    s = lax.axis_index("s")

    @pl.when(s < N // ROWS_PER_TILE)
    def _():
        pltpu.sync_copy(zeros_hbm,
                        accum_sh.at[pl.ds(s * ROWS_PER_TILE, ROWS_PER_TILE)])
    plsc.subcore_barrier()

    per = (NCHUNK + NS - 1) // NS

    def chunk_body(i, _):
        chunk = s * per + i

        @pl.when(chunk < NCHUNK)
        def _():
            base = chunk * C
            pltpu.sync_copy(dst_hbm.at[pl.ds(base, C)], idx_v)
            pltpu.sync_copy(msg_hbm.at[c, pl.ds(base, C)], buf_v)
            pltpu.sync_copy(buf_v, accum_sh.at[idx_v], add=True)
        return 0

    lax.fori_loop(0, per, chunk_body, 0)
    plsc.subcore_barrier()

    @pl.when(s < N // ROWS_PER_TILE)
    def _():
        pltpu.sync_copy(accum_sh.at[pl.ds(s * ROWS_PER_TILE, ROWS_PER_TILE)],
                        agg_hbm.at[c, pl.ds(s * ROWS_PER_TILE, ROWS_PER_TILE)])


_sc_scatter = functools.partial(
    pl.kernel,
    out_type=jax.ShapeDtypeStruct((NC, N, D), jnp.float32),
    mesh=plsc.VectorSubcoreMesh(core_axis_name="c", subcore_axis_name="s",
                                num_cores=NC, num_subcores=NS),
    scratch_types=(pltpu.VMEM_SHARED((N, D), jnp.float32),
                   pltpu.VMEM((C,), jnp.int32),
                   pltpu.VMEM((C, D), jnp.float32),
                   pltpu.SemaphoreType.DMA),
)(_scatter_body)


EB = 2000


def _edge_mlp_body(g1_ref, g2_ref, wp_ref, wq_ref, b1_ref, w2_ref, b2_ref,
                   out_ref):
    pre = (jnp.dot(g1_ref[...], wp_ref[...], preferred_element_type=jnp.float32)
           + jnp.dot(g2_ref[...], wq_ref[...], preferred_element_type=jnp.float32)
           + b1_ref[...])
    h1 = _elu(pre)
    msg = _elu(jnp.dot(h1, w2_ref[...], preferred_element_type=jnp.float32)
               + b2_ref[...])
    out_ref[0] = msg[:, :D]
    out_ref[1] = msg[:, D:]


def _tc_edge(g1, g2, wp, wq, b1, w2, b2):
    return pl.pallas_call(
        _edge_mlp_body,
        grid=(E // EB,),
        in_specs=[
            pl.BlockSpec((EB, D), lambda i: (i, 0)),
            pl.BlockSpec((EB, D), lambda i: (i, 0)),
            pl.BlockSpec((D, 2 * D), lambda i: (0, 0)),
            pl.BlockSpec((D, 2 * D), lambda i: (0, 0)),
            pl.BlockSpec((1, 2 * D), lambda i: (0, 0)),
            pl.BlockSpec((2 * D, 2 * D), lambda i: (0, 0)),
            pl.BlockSpec((1, 2 * D), lambda i: (0, 0)),
        ],
        out_specs=pl.BlockSpec((NC, EB, D), lambda i: (0, i, 0)),
        out_shape=jax.ShapeDtypeStruct((NC, E, D), jnp.float32),
    )(g1, g2, wp, wq, b1, w2, b2)


RB = 2000


def _node_mlp_body(agg_ref, m2w1_ref, m2b1_ref, m2w2_ref, m2b2_ref,
                   mow1_ref, mob1_ref, mow2_ref, mob2_ref,
                   wi1_ref, bi1_ref, h_ref):
    x = jnp.concatenate([agg_ref[0], agg_ref[1]], axis=1)
    u = _elu(jnp.dot(x, m2w1_ref[...], preferred_element_type=jnp.float32)
             + m2b1_ref[...])
    u = _elu(jnp.dot(u, m2w2_ref[...], preferred_element_type=jnp.float32)
             + m2b2_ref[...])
    o = _elu(jnp.dot(u, mow1_ref[...], preferred_element_type=jnp.float32)
             + mob1_ref[...])
    o = _elu(jnp.dot(o, mow2_ref[...], preferred_element_type=jnp.float32)
             + mob2_ref[...])
    o = o[:, D:]
    h_ref[...] = jnp.maximum(
        jnp.dot(o, wi1_ref[...], preferred_element_type=jnp.float32)
        + bi1_ref[...], 0.0)


def _tc_node_mlp(agg, m2W1, m2b1, m2W2, m2b2, moW1, mob1, moW2, mob2, Wi1, bi1):
    full = lambda r, c: pl.BlockSpec((r, c), lambda i: (0, 0))
    return pl.pallas_call(
        _node_mlp_body,
        grid=(N // RB,),
        in_specs=[
            pl.BlockSpec((NC, RB, D), lambda i: (0, i, 0)),
            full(2 * D, 2 * D), full(1, 2 * D),
            full(2 * D, 2 * D), full(1, 2 * D),
            full(2 * D, 2 * D), full(1, 2 * D),
            full(2 * D, 2 * D), full(1, 2 * D),
            full(D, 3 * D), full(1, 3 * D),
        ],
        out_specs=pl.BlockSpec((RB, 3 * D), lambda i: (i, 0)),
        out_shape=jax.ShapeDtypeStruct((N, 3 * D), jnp.float32),
    )(agg, m2W1, m2b1.reshape(1, -1), m2W2, m2b2.reshape(1, -1),
      moW1, mob1.reshape(1, -1), moW2, mob2.reshape(1, -1),
      Wi1, bi1.reshape(1, -1))


FRB = 200


def _final_body(h_ref, w_ref, b_ref, out_ref):
    z = (jnp.dot(h_ref[...], w_ref[...], preferred_element_type=jnp.float32)
         + b_ref[...])
    out_ref[...] = 1.0 / (1.0 + jnp.exp(-z))


def _tc_final(h, Wi2, bi2):
    return pl.pallas_call(
        _final_body,
        grid=(N // FRB,),
        in_specs=[
            pl.BlockSpec((FRB, 3 * D), lambda i: (i, 0)),
            pl.BlockSpec((3 * D, NUM_API), lambda i: (0, 0)),
            pl.BlockSpec((1, NUM_API), lambda i: (0, 0)),
        ],
        out_specs=pl.BlockSpec((FRB, NUM_API), lambda i: (i, 0)),
        out_shape=jax.ShapeDtypeStruct((N, NUM_API), jnp.float32),
    )(h, Wi2, bi2.reshape(1, -1))


def kernel(api_embeds, adjacency_matrix, edge_index,
           Wr1, br1, Wr2, br2,
           m1W1, m1b1, m1W2, m1b2,
           m2W1, m2b1, m2W2, m2b2,
           moW1, mob1, moW2, mob2,
           Wi1, bi1, Wi2, bi2):
    del adjacency_matrix, Wr1, br1, Wr2, br2
    src = edge_index[0]
    dst = edge_index[1]
    WP = m1W1[:D] + m1W1[D:2 * D]
    WQ = m1W1[2 * D:3 * D] + m1W1[3 * D:]

    g1, g2 = _sc_gather(api_embeds, dst, src)
    msg = _tc_edge(g1, g2, WP, WQ, m1b1.reshape(1, -1), m1W2,
                   m1b2.reshape(1, -1))
    zeros = jnp.zeros((ROWS_PER_TILE, D), jnp.float32)
    agg = _sc_scatter(msg, dst, zeros)
    h = _tc_node_mlp(agg, m2W1, m2b1, m2W2, m2b2,
                     moW1, mob1, moW2, mob2, Wi1, bi1)
    return _tc_final(h, Wi2, bi2)


# speedup vs baseline: 2.6319x; 1.0075x over previous
import functools

import jax
import jax.numpy as jnp
from jax import lax
from jax.experimental import pallas as pl
from jax.experimental.pallas import tpu as pltpu
from jax.experimental.pallas import tpu_sc as plsc

D = 128
N = 10000
NUM_API = 10000
E = 160000

NC = 2
NS = 16
C = 128
NCHUNK = E // C


def _elu(x):
    return jnp.where(x > 0, x, jnp.exp(jnp.minimum(x, 0.0)) - 1.0)


def _gather_body(a_hbm, dst_hbm, src_hbm, g1_hbm, g2_hbm,
                 idx1_v, idx2_v, buf1_v, buf2_v, sem1, sem2):
    wid = lax.axis_index("s") * NC + lax.axis_index("c")
    per = (NCHUNK + NC * NS - 1) // (NC * NS)

    def chunk_body(i, _):
        chunk = wid * per + i

        @pl.when(chunk < NCHUNK)
        def _():
            base = chunk * C
            pltpu.sync_copy(dst_hbm.at[pl.ds(base, C)], idx1_v)
            pltpu.sync_copy(src_hbm.at[pl.ds(base, C)], idx2_v)
            cp1 = pltpu.async_copy(a_hbm.at[idx1_v], buf1_v, sem1)
            cp2 = pltpu.async_copy(a_hbm.at[idx2_v], buf2_v, sem2)
            cp1.wait()
            cp2.wait()
            pltpu.sync_copy(buf1_v, g1_hbm.at[pl.ds(base, C)])
            pltpu.sync_copy(buf2_v, g2_hbm.at[pl.ds(base, C)])
        return 0

    lax.fori_loop(0, per, chunk_body, 0)


_sc_gather = functools.partial(
    pl.kernel,
    out_type=(jax.ShapeDtypeStruct((E, D), jnp.float32),
              jax.ShapeDtypeStruct((E, D), jnp.float32)),
    mesh=plsc.VectorSubcoreMesh(core_axis_name="c", subcore_axis_name="s",
                                num_cores=NC, num_subcores=NS),
    scratch_types=(pltpu.VMEM((C,), jnp.int32),
                   pltpu.VMEM((C,), jnp.int32),
                   pltpu.VMEM((C, D), jnp.float32),
                   pltpu.VMEM((C, D), jnp.float32),
                   pltpu.SemaphoreType.DMA,
                   pltpu.SemaphoreType.DMA),
)(_gather_body)


ROWS_PER_TILE = 1000


def _scatter_body(msg_hbm, dst_hbm, zeros_hbm, agg_hbm,
                  accum_sh, idx_v, buf_v, sem):
    c = lax.axis_index("c")
    s = lax.axis_index("s")

    @pl.when(s < N // ROWS_PER_TILE)
    def _():
        pltpu.sync_copy(zeros_hbm,
                        accum_sh.at[pl.ds(s * ROWS_PER_TILE, ROWS_PER_TILE)])
    plsc.subcore_barrier()

    per = (NCHUNK + NS - 1) // NS

    def chunk_body(i, _):
        chunk = s * per + i

        @pl.when(chunk < NCHUNK)
        def _():
            base = chunk * C
            pltpu.sync_copy(dst_hbm.at[pl.ds(base, C)], idx_v)
            pltpu.sync_copy(msg_hbm.at[c, pl.ds(base, C)], buf_v)
            pltpu.sync_copy(buf_v, accum_sh.at[idx_v], add=True)
        return 0

    lax.fori_loop(0, per, chunk_body, 0)
    plsc.subcore_barrier()

    @pl.when(s < N // ROWS_PER_TILE)
    def _():
        pltpu.sync_copy(accum_sh.at[pl.ds(s * ROWS_PER_TILE, ROWS_PER_TILE)],
                        agg_hbm.at[c, pl.ds(s * ROWS_PER_TILE, ROWS_PER_TILE)])


_sc_scatter = functools.partial(
    pl.kernel,
    out_type=jax.ShapeDtypeStruct((NC, N, D), jnp.float32),
    mesh=plsc.VectorSubcoreMesh(core_axis_name="c", subcore_axis_name="s",
                                num_cores=NC, num_subcores=NS),
    scratch_types=(pltpu.VMEM_SHARED((N, D), jnp.float32),
                   pltpu.VMEM((C,), jnp.int32),
                   pltpu.VMEM((C, D), jnp.float32),
                   pltpu.SemaphoreType.DMA),
)(_scatter_body)


EB = 2000


def _edge_mlp_body(g1_ref, g2_ref, wp_ref, wq_ref, b1_ref, w2_ref, b2_ref,
                   out_ref):
    bf = jnp.bfloat16
    pre = (jnp.dot(g1_ref[...].astype(bf), wp_ref[...],
                   preferred_element_type=jnp.float32)
           + jnp.dot(g2_ref[...].astype(bf), wq_ref[...],
                     preferred_element_type=jnp.float32)
           + b1_ref[...])
    h1 = _elu(pre)
    msg = _elu(jnp.dot(h1.astype(bf), w2_ref[...],
                       preferred_element_type=jnp.float32)
               + b2_ref[...])
    out_ref[0] = msg[:, :D]
    out_ref[1] = msg[:, D:]


def _tc_edge(g1, g2, wp, wq, b1, w2, b2):
    return pl.pallas_call(
        _edge_mlp_body,
        grid=(E // EB,),
        in_specs=[
            pl.BlockSpec((EB, D), lambda i: (i, 0)),
            pl.BlockSpec((EB, D), lambda i: (i, 0)),
            pl.BlockSpec((D, 2 * D), lambda i: (0, 0)),
            pl.BlockSpec((D, 2 * D), lambda i: (0, 0)),
            pl.BlockSpec((1, 2 * D), lambda i: (0, 0)),
            pl.BlockSpec((2 * D, 2 * D), lambda i: (0, 0)),
            pl.BlockSpec((1, 2 * D), lambda i: (0, 0)),
        ],
        out_specs=pl.BlockSpec((NC, EB, D), lambda i: (0, i, 0)),
        out_shape=jax.ShapeDtypeStruct((NC, E, D), jnp.float32),
    )(g1, g2, wp, wq, b1, w2, b2)


RB = 2000


def _node_mlp_body(agg_ref, m2w1_ref, m2b1_ref, m2w2_ref, m2b2_ref,
                   mow1_ref, mob1_ref, mow2_ref, mob2_ref,
                   wi1_ref, bi1_ref, h_ref):
    bf = jnp.bfloat16
    x = jnp.concatenate([agg_ref[0], agg_ref[1]], axis=1)
    u = _elu(jnp.dot(x.astype(bf), m2w1_ref[...],
                     preferred_element_type=jnp.float32) + m2b1_ref[...])
    u = _elu(jnp.dot(u.astype(bf), m2w2_ref[...],
                     preferred_element_type=jnp.float32) + m2b2_ref[...])
    o = _elu(jnp.dot(u.astype(bf), mow1_ref[...],
                     preferred_element_type=jnp.float32) + mob1_ref[...])
    o = _elu(jnp.dot(o.astype(bf), mow2_ref[...],
                     preferred_element_type=jnp.float32) + mob2_ref[...])
    o = o[:, D:]
    h_ref[...] = jnp.maximum(
        jnp.dot(o.astype(bf), wi1_ref[...], preferred_element_type=jnp.float32)
        + bi1_ref[...], 0.0).astype(bf)


def _tc_node_mlp(agg, m2W1, m2b1, m2W2, m2b2, moW1, mob1, moW2, mob2, Wi1, bi1):
    full = lambda r, c: pl.BlockSpec((r, c), lambda i: (0, 0))
    return pl.pallas_call(
        _node_mlp_body,
        grid=(N // RB,),
        in_specs=[
            pl.BlockSpec((NC, RB, D), lambda i: (0, i, 0)),
            full(2 * D, 2 * D), full(1, 2 * D),
            full(2 * D, 2 * D), full(1, 2 * D),
            full(2 * D, 2 * D), full(1, 2 * D),
            full(2 * D, 2 * D), full(1, 2 * D),
            full(D, 3 * D), full(1, 3 * D),
        ],
        out_specs=pl.BlockSpec((RB, 3 * D), lambda i: (i, 0)),
        out_shape=jax.ShapeDtypeStruct((N, 3 * D), jnp.bfloat16),
    )(agg, m2W1, m2b1.reshape(1, -1), m2W2, m2b2.reshape(1, -1),
      moW1, mob1.reshape(1, -1), moW2, mob2.reshape(1, -1),
      Wi1, bi1.reshape(1, -1))


FRB = 200


def _final_body(h_ref, w_ref, b_ref, out_ref):
    z = (jnp.dot(h_ref[...], w_ref[...], preferred_element_type=jnp.float32)
         + b_ref[...])
    out_ref[...] = 1.0 / (1.0 + jnp.exp(-z))


def _tc_final(h, Wi2, bi2):
    return pl.pallas_call(
        _final_body,
        grid=(N // FRB,),
        in_specs=[
            pl.BlockSpec((FRB, 3 * D), lambda i: (i, 0)),
            pl.BlockSpec((3 * D, NUM_API), lambda i: (0, 0)),
            pl.BlockSpec((1, NUM_API), lambda i: (0, 0)),
        ],
        out_specs=pl.BlockSpec((FRB, NUM_API), lambda i: (i, 0)),
        out_shape=jax.ShapeDtypeStruct((N, NUM_API), jnp.float32),
    )(h, Wi2, bi2.reshape(1, -1))


def kernel(api_embeds, adjacency_matrix, edge_index,
           Wr1, br1, Wr2, br2,
           m1W1, m1b1, m1W2, m1b2,
           m2W1, m2b1, m2W2, m2b2,
           moW1, mob1, moW2, mob2,
           Wi1, bi1, Wi2, bi2):
    del adjacency_matrix, Wr1, br1, Wr2, br2
    src = edge_index[0]
    dst = edge_index[1]
    bf = jnp.bfloat16
    WP = (m1W1[:D] + m1W1[D:2 * D]).astype(bf)
    WQ = (m1W1[2 * D:3 * D] + m1W1[3 * D:]).astype(bf)

    g1, g2 = _sc_gather(api_embeds, dst, src)
    msg = _tc_edge(g1, g2, WP, WQ, m1b1.reshape(1, -1), m1W2.astype(bf),
                   m1b2.reshape(1, -1))
    zeros = jnp.zeros((ROWS_PER_TILE, D), jnp.float32)
    agg = _sc_scatter(msg, dst, zeros)
    h = _tc_node_mlp(agg, m2W1.astype(bf), m2b1, m2W2.astype(bf), m2b2,
                     moW1.astype(bf), mob1, moW2.astype(bf), mob2,
                     Wi1.astype(bf), bi1)
    return _tc_final(h, Wi2.astype(bf), bi2)


# speedup vs baseline: 3.2253x; 1.2254x over previous
import functools

import jax
import jax.numpy as jnp
from jax import lax
from jax.experimental import pallas as pl
from jax.experimental.pallas import tpu as pltpu
from jax.experimental.pallas import tpu_sc as plsc

D = 128
N = 10000
NUM_API = 10000
E = 160000

NC = 2
NS = 16
C = 128
NCHUNK = E // C
GPER = 40
SPER = 80
NPAD = 1280


def _elu(x):
    return jnp.where(x > 0, x, jnp.exp(jnp.minimum(x, 0.0)) - 1.0)


def _gather_body(a_hbm, dstp_hbm, srcp_hbm, g1_hbm, g2_hbm,
                 idxd_v, idxs_v, b1_v, b2_v, gsem, wsem):
    wid = lax.axis_index("s") * NC + lax.axis_index("c")
    first = wid * GPER

    pltpu.sync_copy(dstp_hbm.at[pl.ds(first, GPER)], idxd_v)
    pltpu.sync_copy(srcp_hbm.at[pl.ds(first, GPER)], idxs_v)

    def valid(i):
        return jnp.logical_and(i < GPER, first + i < NCHUNK)

    def pair(i2, _):
        i = i2 * 2
        j = i + 1

        @pl.when(valid(i))
        def _():
            cp1 = pltpu.async_copy(a_hbm.at[idxd_v.at[i]], b1_v.at[0], gsem)
            cp2 = pltpu.async_copy(a_hbm.at[idxs_v.at[i]], b2_v.at[0], gsem)

            @pl.when(valid(j))
            def _():
                cp3 = pltpu.async_copy(a_hbm.at[idxd_v.at[j]], b1_v.at[1], gsem)
                cp4 = pltpu.async_copy(a_hbm.at[idxs_v.at[j]], b2_v.at[1], gsem)
                cp1.wait()
                cp2.wait()
                base = (first + i) * C
                w1 = pltpu.async_copy(b1_v.at[0], g1_hbm.at[pl.ds(base, C)], wsem)
                w2 = pltpu.async_copy(b2_v.at[0], g2_hbm.at[pl.ds(base, C)], wsem)
                cp3.wait()
                cp4.wait()
                basej = (first + j) * C
                w3 = pltpu.async_copy(b1_v.at[1], g1_hbm.at[pl.ds(basej, C)], wsem)
                w4 = pltpu.async_copy(b2_v.at[1], g2_hbm.at[pl.ds(basej, C)], wsem)
                w1.wait()
                w2.wait()
                w3.wait()
                w4.wait()

            @pl.when(jnp.logical_not(valid(j)))
            def _():
                cp1.wait()
                cp2.wait()
                base = (first + i) * C
                w1 = pltpu.async_copy(b1_v.at[0], g1_hbm.at[pl.ds(base, C)], wsem)
                w2 = pltpu.async_copy(b2_v.at[0], g2_hbm.at[pl.ds(base, C)], wsem)
                w1.wait()
                w2.wait()
        return 0

    lax.fori_loop(0, GPER // 2, pair, 0)


_sc_gather = functools.partial(
    pl.kernel,
    out_type=(jax.ShapeDtypeStruct((E, D), jnp.float32),
              jax.ShapeDtypeStruct((E, D), jnp.float32)),
    mesh=plsc.VectorSubcoreMesh(core_axis_name="c", subcore_axis_name="s",
                                num_cores=NC, num_subcores=NS),
    scratch_types=(pltpu.VMEM((GPER, C), jnp.int32),
                   pltpu.VMEM((GPER, C), jnp.int32),
                   pltpu.VMEM((2, C, D), jnp.float32),
                   pltpu.VMEM((2, C, D), jnp.float32),
                   pltpu.SemaphoreType.DMA,
                   pltpu.SemaphoreType.DMA),
)(_gather_body)


ROWS_PER_TILE = 1000


def _scatter_body(msg_hbm, dstp_hbm, zeros_hbm, agg_hbm,
                  accum_sh, idx_v, mbuf_v, lsem, asem):
    c = lax.axis_index("c")
    s = lax.axis_index("s")
    first = s * SPER

    @pl.when(s < N // ROWS_PER_TILE)
    def _():
        pltpu.sync_copy(zeros_hbm,
                        accum_sh.at[pl.ds(s * ROWS_PER_TILE, ROWS_PER_TILE)])

    pltpu.sync_copy(dstp_hbm.at[pl.ds(first, SPER)], idx_v)
    plsc.subcore_barrier()

    def valid(i):
        return jnp.logical_and(i < SPER, first + i < NCHUNK)

    def pair(i2, _):
        i = i2 * 2
        j = i + 1

        @pl.when(valid(i))
        def _():
            l1 = pltpu.async_copy(msg_hbm.at[c, pl.ds((first + i) * C, C)],
                                  mbuf_v.at[0], lsem)

            @pl.when(valid(j))
            def _():
                l2 = pltpu.async_copy(msg_hbm.at[c, pl.ds((first + j) * C, C)],
                                      mbuf_v.at[1], lsem)
                l1.wait()
                a1 = pltpu.async_copy(mbuf_v.at[0], accum_sh.at[idx_v.at[i]],
                                      asem, add=True)
                l2.wait()
                a2 = pltpu.async_copy(mbuf_v.at[1], accum_sh.at[idx_v.at[j]],
                                      asem, add=True)
                a1.wait()
                a2.wait()

            @pl.when(jnp.logical_not(valid(j)))
            def _():
                l1.wait()
                a1 = pltpu.async_copy(mbuf_v.at[0], accum_sh.at[idx_v.at[i]],
                                      asem, add=True)
                a1.wait()
        return 0

    lax.fori_loop(0, SPER // 2, pair, 0)
    plsc.subcore_barrier()

    @pl.when(s < N // ROWS_PER_TILE)
    def _():
        pltpu.sync_copy(accum_sh.at[pl.ds(s * ROWS_PER_TILE, ROWS_PER_TILE)],
                        agg_hbm.at[c, pl.ds(s * ROWS_PER_TILE, ROWS_PER_TILE)])


_sc_scatter = functools.partial(
    pl.kernel,
    out_type=jax.ShapeDtypeStruct((NC, N, D), jnp.float32),
    mesh=plsc.VectorSubcoreMesh(core_axis_name="c", subcore_axis_name="s",
                                num_cores=NC, num_subcores=NS),
    scratch_types=(pltpu.VMEM_SHARED((N, D), jnp.float32),
                   pltpu.VMEM((SPER, C), jnp.int32),
                   pltpu.VMEM((2, C, D), jnp.float32),
                   pltpu.SemaphoreType.DMA,
                   pltpu.SemaphoreType.DMA),
)(_scatter_body)


EB = 2000


def _edge_mlp_body(g1_ref, g2_ref, wp_ref, wq_ref, b1_ref, w2_ref, b2_ref,
                   out_ref):
    bf = jnp.bfloat16
    pre = (jnp.dot(g1_ref[...].astype(bf), wp_ref[...],
                   preferred_element_type=jnp.float32)
           + jnp.dot(g2_ref[...].astype(bf), wq_ref[...],
                     preferred_element_type=jnp.float32)
           + b1_ref[...])
    h1 = _elu(pre)
    msg = _elu(jnp.dot(h1.astype(bf), w2_ref[...],
                       preferred_element_type=jnp.float32)
               + b2_ref[...])
    out_ref[0] = msg[:, :D]
    out_ref[1] = msg[:, D:]


def _tc_edge(g1, g2, wp, wq, b1, w2, b2):
    return pl.pallas_call(
        _edge_mlp_body,
        grid=(E // EB,),
        in_specs=[
            pl.BlockSpec((EB, D), lambda i: (i, 0)),
            pl.BlockSpec((EB, D), lambda i: (i, 0)),
            pl.BlockSpec((D, 2 * D), lambda i: (0, 0)),
            pl.BlockSpec((D, 2 * D), lambda i: (0, 0)),
            pl.BlockSpec((1, 2 * D), lambda i: (0, 0)),
            pl.BlockSpec((2 * D, 2 * D), lambda i: (0, 0)),
            pl.BlockSpec((1, 2 * D), lambda i: (0, 0)),
        ],
        out_specs=pl.BlockSpec((NC, EB, D), lambda i: (0, i, 0)),
        out_shape=jax.ShapeDtypeStruct((NC, E, D), jnp.float32),
    )(g1, g2, wp, wq, b1, w2, b2)


RB = 2000


def _node_mlp_body(agg_ref, m2w1_ref, m2b1_ref, m2w2_ref, m2b2_ref,
                   mow1_ref, mob1_ref, mow2_ref, mob2_ref,
                   wi1_ref, bi1_ref, h_ref):
    bf = jnp.bfloat16
    x = jnp.concatenate([agg_ref[0], agg_ref[1]], axis=1)
    u = _elu(jnp.dot(x.astype(bf), m2w1_ref[...],
                     preferred_element_type=jnp.float32) + m2b1_ref[...])
    u = _elu(jnp.dot(u.astype(bf), m2w2_ref[...],
                     preferred_element_type=jnp.float32) + m2b2_ref[...])
    o = _elu(jnp.dot(u.astype(bf), mow1_ref[...],
                     preferred_element_type=jnp.float32) + mob1_ref[...])
    o = _elu(jnp.dot(o.astype(bf), mow2_ref[...],
                     preferred_element_type=jnp.float32) + mob2_ref[...])
    o = o[:, D:]
    h_ref[...] = jnp.maximum(
        jnp.dot(o.astype(bf), wi1_ref[...], preferred_element_type=jnp.float32)
        + bi1_ref[...], 0.0).astype(bf)


def _tc_node_mlp(agg, m2W1, m2b1, m2W2, m2b2, moW1, mob1, moW2, mob2, Wi1, bi1):
    full = lambda r, c: pl.BlockSpec((r, c), lambda i: (0, 0))
    return pl.pallas_call(
        _node_mlp_body,
        grid=(N // RB,),
        in_specs=[
            pl.BlockSpec((NC, RB, D), lambda i: (0, i, 0)),
            full(2 * D, 2 * D), full(1, 2 * D),
            full(2 * D, 2 * D), full(1, 2 * D),
            full(2 * D, 2 * D), full(1, 2 * D),
            full(2 * D, 2 * D), full(1, 2 * D),
            full(D, 3 * D), full(1, 3 * D),
        ],
        out_specs=pl.BlockSpec((RB, 3 * D), lambda i: (i, 0)),
        out_shape=jax.ShapeDtypeStruct((N, 3 * D), jnp.bfloat16),
    )(agg, m2W1, m2b1.reshape(1, -1), m2W2, m2b2.reshape(1, -1),
      moW1, mob1.reshape(1, -1), moW2, mob2.reshape(1, -1),
      Wi1, bi1.reshape(1, -1))


FRB = 200


def _final_body(h_ref, w_ref, b_ref, out_ref):
    z = (jnp.dot(h_ref[...], w_ref[...], preferred_element_type=jnp.float32)
         + b_ref[...])
    out_ref[...] = 1.0 / (1.0 + jnp.exp(-z))


def _tc_final(h, Wi2, bi2):
    return pl.pallas_call(
        _final_body,
        grid=(N // FRB,),
        in_specs=[
            pl.BlockSpec((FRB, 3 * D), lambda i: (i, 0)),
            pl.BlockSpec((3 * D, NUM_API), lambda i: (0, 0)),
            pl.BlockSpec((1, NUM_API), lambda i: (0, 0)),
        ],
        out_specs=pl.BlockSpec((FRB, NUM_API), lambda i: (i, 0)),
        out_shape=jax.ShapeDtypeStruct((N, NUM_API), jnp.float32),
    )(h, Wi2, bi2.reshape(1, -1))


def kernel(api_embeds, adjacency_matrix, edge_index,
           Wr1, br1, Wr2, br2,
           m1W1, m1b1, m1W2, m1b2,
           m2W1, m2b1, m2W2, m2b2,
           moW1, mob1, moW2, mob2,
           Wi1, bi1, Wi2, bi2):
    del adjacency_matrix, Wr1, br1, Wr2, br2
    src = edge_index[0]
    dst = edge_index[1]
    dstp = jnp.pad(dst.reshape(NCHUNK, C), ((0, NPAD - NCHUNK), (0, 0)))
    srcp = jnp.pad(src.reshape(NCHUNK, C), ((0, NPAD - NCHUNK), (0, 0)))

    bf = jnp.bfloat16
    WP = (m1W1[:D] + m1W1[D:2 * D]).astype(bf)
    WQ = (m1W1[2 * D:3 * D] + m1W1[3 * D:]).astype(bf)

    g1, g2 = _sc_gather(api_embeds, dstp, srcp)
    msg = _tc_edge(g1, g2, WP, WQ, m1b1.reshape(1, -1), m1W2.astype(bf),
                   m1b2.reshape(1, -1))
    zeros = jnp.zeros((ROWS_PER_TILE, D), jnp.float32)
    agg = _sc_scatter(msg, dstp, zeros)
    h = _tc_node_mlp(agg, m2W1.astype(bf), m2b1, m2W2.astype(bf), m2b2,
                     moW1.astype(bf), mob1, moW2.astype(bf), mob2,
                     Wi1.astype(bf), bi1)
    return _tc_final(h, Wi2.astype(bf), bi2)


# speedup vs baseline: 3.3612x; 1.0422x over previous
import functools

import jax
import jax.numpy as jnp
from jax import lax
from jax.experimental import pallas as pl
from jax.experimental.pallas import tpu as pltpu
from jax.experimental.pallas import tpu_sc as plsc

D = 128
N = 10000
NUM_API = 10000
E = 160000

NC = 2
NS = 16
C = 128
NCHUNK = E // C
GPER = 40
SPER = 80
NPAD = 1280


def _elu(x):
    return jnp.where(x > 0, x, jnp.exp(jnp.minimum(x, 0.0)) - 1.0)


def _gather_body(a_hbm, dstp_hbm, srcp_hbm, g1_hbm, g2_hbm,
                 idxd_v, idxs_v, b1_v, b2_v, gsem, wsem):
    wid = lax.axis_index("s") * NC + lax.axis_index("c")
    first = wid * GPER

    pltpu.sync_copy(dstp_hbm.at[pl.ds(first, GPER)], idxd_v)
    pltpu.sync_copy(srcp_hbm.at[pl.ds(first, GPER)], idxs_v)

    def valid(i):
        return jnp.logical_and(i < GPER, first + i < NCHUNK)

    def pair(i2, _):
        i = i2 * 2
        j = i + 1

        @pl.when(valid(i))
        def _():
            cp1 = pltpu.async_copy(a_hbm.at[idxd_v.at[i]], b1_v.at[0], gsem)
            cp2 = pltpu.async_copy(a_hbm.at[idxs_v.at[i]], b2_v.at[0], gsem)

            @pl.when(valid(j))
            def _():
                cp3 = pltpu.async_copy(a_hbm.at[idxd_v.at[j]], b1_v.at[1], gsem)
                cp4 = pltpu.async_copy(a_hbm.at[idxs_v.at[j]], b2_v.at[1], gsem)
                cp1.wait()
                cp2.wait()
                base = (first + i) * C
                w1 = pltpu.async_copy(b1_v.at[0], g1_hbm.at[pl.ds(base, C)], wsem)
                w2 = pltpu.async_copy(b2_v.at[0], g2_hbm.at[pl.ds(base, C)], wsem)
                cp3.wait()
                cp4.wait()
                basej = (first + j) * C
                w3 = pltpu.async_copy(b1_v.at[1], g1_hbm.at[pl.ds(basej, C)], wsem)
                w4 = pltpu.async_copy(b2_v.at[1], g2_hbm.at[pl.ds(basej, C)], wsem)
                w1.wait()
                w2.wait()
                w3.wait()
                w4.wait()

            @pl.when(jnp.logical_not(valid(j)))
            def _():
                cp1.wait()
                cp2.wait()
                base = (first + i) * C
                w1 = pltpu.async_copy(b1_v.at[0], g1_hbm.at[pl.ds(base, C)], wsem)
                w2 = pltpu.async_copy(b2_v.at[0], g2_hbm.at[pl.ds(base, C)], wsem)
                w1.wait()
                w2.wait()
        return 0

    lax.fori_loop(0, GPER // 2, pair, 0)


_sc_gather = functools.partial(
    pl.kernel,
    out_type=(jax.ShapeDtypeStruct((E, D), jnp.float32),
              jax.ShapeDtypeStruct((E, D), jnp.float32)),
    mesh=plsc.VectorSubcoreMesh(core_axis_name="c", subcore_axis_name="s",
                                num_cores=NC, num_subcores=NS),
    scratch_types=(pltpu.VMEM((GPER, C), jnp.int32),
                   pltpu.VMEM((GPER, C), jnp.int32),
                   pltpu.VMEM((2, C, D), jnp.float32),
                   pltpu.VMEM((2, C, D), jnp.float32),
                   pltpu.SemaphoreType.DMA,
                   pltpu.SemaphoreType.DMA),
)(_gather_body)


ROWS_PER_TILE = 1000


def _scatter_body(msg_hbm, dstp_hbm, zeros_hbm, agg_hbm,
                  accum_sh, idx_v, mbuf_v, lsem, asem):
    c = lax.axis_index("c")
    s = lax.axis_index("s")
    first = s * SPER

    @pl.when(s < N // ROWS_PER_TILE)
    def _():
        pltpu.sync_copy(zeros_hbm,
                        accum_sh.at[pl.ds(s * ROWS_PER_TILE, ROWS_PER_TILE)])

    pltpu.sync_copy(dstp_hbm.at[pl.ds(first, SPER)], idx_v)
    plsc.subcore_barrier()

    def valid(i):
        return jnp.logical_and(i < SPER, first + i < NCHUNK)

    def pair(i2, _):
        i = i2 * 2
        j = i + 1

        @pl.when(valid(i))
        def _():
            l1 = pltpu.async_copy(msg_hbm.at[c, pl.ds((first + i) * C, C)],
                                  mbuf_v.at[0], lsem)

            @pl.when(valid(j))
            def _():
                l2 = pltpu.async_copy(msg_hbm.at[c, pl.ds((first + j) * C, C)],
                                      mbuf_v.at[1], lsem)
                l1.wait()
                a1 = pltpu.async_copy(mbuf_v.at[0], accum_sh.at[idx_v.at[i]],
                                      asem, add=True)
                l2.wait()
                a2 = pltpu.async_copy(mbuf_v.at[1], accum_sh.at[idx_v.at[j]],
                                      asem, add=True)
                a1.wait()
                a2.wait()

            @pl.when(jnp.logical_not(valid(j)))
            def _():
                l1.wait()
                a1 = pltpu.async_copy(mbuf_v.at[0], accum_sh.at[idx_v.at[i]],
                                      asem, add=True)
                a1.wait()
        return 0

    lax.fori_loop(0, SPER // 2, pair, 0)
    plsc.subcore_barrier()

    @pl.when(s < N // ROWS_PER_TILE)
    def _():
        pltpu.sync_copy(accum_sh.at[pl.ds(s * ROWS_PER_TILE, ROWS_PER_TILE)],
                        agg_hbm.at[c, pl.ds(s * ROWS_PER_TILE, ROWS_PER_TILE)])


_sc_scatter = functools.partial(
    pl.kernel,
    out_type=jax.ShapeDtypeStruct((NC, N, D), jnp.float32),
    mesh=plsc.VectorSubcoreMesh(core_axis_name="c", subcore_axis_name="s",
                                num_cores=NC, num_subcores=NS),
    scratch_types=(pltpu.VMEM_SHARED((N, D), jnp.float32),
                   pltpu.VMEM((SPER, C), jnp.int32),
                   pltpu.VMEM((2, C, D), jnp.float32),
                   pltpu.SemaphoreType.DMA,
                   pltpu.SemaphoreType.DMA),
)(_scatter_body)


EB = 4000


def _edge_mlp_body(g1_ref, g2_ref, wp_ref, wq_ref, b1_ref, w2_ref, b2_ref,
                   out_ref):
    bf = jnp.bfloat16
    pre = (jnp.dot(g1_ref[...].astype(bf), wp_ref[...],
                   preferred_element_type=jnp.float32)
           + jnp.dot(g2_ref[...].astype(bf), wq_ref[...],
                     preferred_element_type=jnp.float32)
           + b1_ref[...])
    h1 = _elu(pre)
    msg = _elu(jnp.dot(h1.astype(bf), w2_ref[...],
                       preferred_element_type=jnp.float32)
               + b2_ref[...])
    out_ref[0] = msg[:, :D]
    out_ref[1] = msg[:, D:]


def _tc_edge(g1, g2, wp, wq, b1, w2, b2):
    return pl.pallas_call(
        _edge_mlp_body,
        grid=(E // EB,),
        in_specs=[
            pl.BlockSpec((EB, D), lambda i: (i, 0)),
            pl.BlockSpec((EB, D), lambda i: (i, 0)),
            pl.BlockSpec((D, 2 * D), lambda i: (0, 0)),
            pl.BlockSpec((D, 2 * D), lambda i: (0, 0)),
            pl.BlockSpec((1, 2 * D), lambda i: (0, 0)),
            pl.BlockSpec((2 * D, 2 * D), lambda i: (0, 0)),
            pl.BlockSpec((1, 2 * D), lambda i: (0, 0)),
        ],
        out_specs=pl.BlockSpec((NC, EB, D), lambda i: (0, i, 0)),
        out_shape=jax.ShapeDtypeStruct((NC, E, D), jnp.float32),
    )(g1, g2, wp, wq, b1, w2, b2)


RB = 2000


def _node_mlp_body(agg_ref, m2w1_ref, m2b1_ref, m2w2_ref, m2b2_ref,
                   mow1_ref, mob1_ref, mow2_ref, mob2_ref,
                   wi1_ref, bi1_ref, h_ref):
    bf = jnp.bfloat16
    x = jnp.concatenate([agg_ref[0], agg_ref[1]], axis=1)
    u = _elu(jnp.dot(x.astype(bf), m2w1_ref[...],
                     preferred_element_type=jnp.float32) + m2b1_ref[...])
    u = _elu(jnp.dot(u.astype(bf), m2w2_ref[...],
                     preferred_element_type=jnp.float32) + m2b2_ref[...])
    o = _elu(jnp.dot(u.astype(bf), mow1_ref[...],
                     preferred_element_type=jnp.float32) + mob1_ref[...])
    o = _elu(jnp.dot(o.astype(bf), mow2_ref[...],
                     preferred_element_type=jnp.float32) + mob2_ref[...])
    o = o[:, D:]
    h_ref[...] = jnp.maximum(
        jnp.dot(o.astype(bf), wi1_ref[...], preferred_element_type=jnp.float32)
        + bi1_ref[...], 0.0).astype(bf)


def _tc_node_mlp(agg, m2W1, m2b1, m2W2, m2b2, moW1, mob1, moW2, mob2, Wi1, bi1):
    full = lambda r, c: pl.BlockSpec((r, c), lambda i: (0, 0))
    return pl.pallas_call(
        _node_mlp_body,
        grid=(N // RB,),
        in_specs=[
            pl.BlockSpec((NC, RB, D), lambda i: (0, i, 0)),
            full(2 * D, 2 * D), full(1, 2 * D),
            full(2 * D, 2 * D), full(1, 2 * D),
            full(2 * D, 2 * D), full(1, 2 * D),
            full(2 * D, 2 * D), full(1, 2 * D),
            full(D, 3 * D), full(1, 3 * D),
        ],
        out_specs=pl.BlockSpec((RB, 3 * D), lambda i: (i, 0)),
        out_shape=jax.ShapeDtypeStruct((N, 3 * D), jnp.bfloat16),
    )(agg, m2W1, m2b1.reshape(1, -1), m2W2, m2b2.reshape(1, -1),
      moW1, mob1.reshape(1, -1), moW2, mob2.reshape(1, -1),
      Wi1, bi1.reshape(1, -1))


FRB = 200


def _final_body(h_ref, w_ref, b_ref, out_ref):
    z = (jnp.dot(h_ref[...], w_ref[...], preferred_element_type=jnp.float32)
         + b_ref[...])
    out_ref[...] = 0.5 + 0.5 * jnp.tanh(0.5 * z)


def _tc_final(h, Wi2, bi2):
    return pl.pallas_call(
        _final_body,
        grid=(N // FRB,),
        in_specs=[
            pl.BlockSpec((FRB, 3 * D), lambda i: (i, 0)),
            pl.BlockSpec((3 * D, NUM_API), lambda i: (0, 0)),
            pl.BlockSpec((1, NUM_API), lambda i: (0, 0)),
        ],
        out_specs=pl.BlockSpec((FRB, NUM_API), lambda i: (i, 0)),
        out_shape=jax.ShapeDtypeStruct((N, NUM_API), jnp.float32),
    )(h, Wi2, bi2.reshape(1, -1))


def kernel(api_embeds, adjacency_matrix, edge_index,
           Wr1, br1, Wr2, br2,
           m1W1, m1b1, m1W2, m1b2,
           m2W1, m2b1, m2W2, m2b2,
           moW1, mob1, moW2, mob2,
           Wi1, bi1, Wi2, bi2):
    del adjacency_matrix, Wr1, br1, Wr2, br2
    src = edge_index[0]
    dst = edge_index[1]
    dstp = jnp.pad(dst.reshape(NCHUNK, C), ((0, NPAD - NCHUNK), (0, 0)))
    srcp = jnp.pad(src.reshape(NCHUNK, C), ((0, NPAD - NCHUNK), (0, 0)))

    bf = jnp.bfloat16
    WP = (m1W1[:D] + m1W1[D:2 * D]).astype(bf)
    WQ = (m1W1[2 * D:3 * D] + m1W1[3 * D:]).astype(bf)

    g1, g2 = _sc_gather(api_embeds, dstp, srcp)
    msg = _tc_edge(g1, g2, WP, WQ, m1b1.reshape(1, -1), m1W2.astype(bf),
                   m1b2.reshape(1, -1))
    zeros = jnp.zeros((ROWS_PER_TILE, D), jnp.float32)
    agg = _sc_scatter(msg, dstp, zeros)
    h = _tc_node_mlp(agg, m2W1.astype(bf), m2b1, m2W2.astype(bf), m2b2,
                     moW1.astype(bf), mob1, moW2.astype(bf), mob2,
                     Wi1.astype(bf), bi1)
    return _tc_final(h, Wi2.astype(bf), bi2)


# speedup vs baseline: 3.5443x; 1.0545x over previous
import functools

import jax
import jax.numpy as jnp
from jax import lax
from jax.experimental import pallas as pl
from jax.experimental.pallas import tpu as pltpu
from jax.experimental.pallas import tpu_sc as plsc

D = 128
N = 10000
NUM_API = 10000
E = 160000

NC = 2
NS = 16
C = 128
EHALF = E // 2
NCHUNK = EHALF // C
GPER = 24
SPER = 40
NPAD = 768


def _elu(x):
    return jnp.where(x > 0, x, jnp.exp(jnp.minimum(x, 0.0)) - 1.0)


def _gather_body(a_hbm, dstp_hbm, srcp_hbm, g1_hbm, g2_hbm,
                 idxd_v, idxs_v, b1_v, b2_v, gsem, wsem):
    wid = lax.axis_index("s") * NC + lax.axis_index("c")
    first = wid * GPER

    pltpu.sync_copy(dstp_hbm.at[pl.ds(first, GPER)], idxd_v)
    pltpu.sync_copy(srcp_hbm.at[pl.ds(first, GPER)], idxs_v)

    def valid(i):
        return jnp.logical_and(i < GPER, first + i < NCHUNK)

    def pair(i2, _):
        i = i2 * 2
        j = i + 1

        @pl.when(valid(i))
        def _():
            cp1 = pltpu.async_copy(a_hbm.at[idxd_v.at[i]], b1_v.at[0], gsem)
            cp2 = pltpu.async_copy(a_hbm.at[idxs_v.at[i]], b2_v.at[0], gsem)

            @pl.when(valid(j))
            def _():
                cp3 = pltpu.async_copy(a_hbm.at[idxd_v.at[j]], b1_v.at[1], gsem)
                cp4 = pltpu.async_copy(a_hbm.at[idxs_v.at[j]], b2_v.at[1], gsem)
                cp1.wait()
                cp2.wait()
                base = (first + i) * C
                w1 = pltpu.async_copy(b1_v.at[0], g1_hbm.at[pl.ds(base, C)], wsem)
                w2 = pltpu.async_copy(b2_v.at[0], g2_hbm.at[pl.ds(base, C)], wsem)
                cp3.wait()
                cp4.wait()
                basej = (first + j) * C
                w3 = pltpu.async_copy(b1_v.at[1], g1_hbm.at[pl.ds(basej, C)], wsem)
                w4 = pltpu.async_copy(b2_v.at[1], g2_hbm.at[pl.ds(basej, C)], wsem)
                w1.wait()
                w2.wait()
                w3.wait()
                w4.wait()

            @pl.when(jnp.logical_not(valid(j)))
            def _():
                cp1.wait()
                cp2.wait()
                base = (first + i) * C
                w1 = pltpu.async_copy(b1_v.at[0], g1_hbm.at[pl.ds(base, C)], wsem)
                w2 = pltpu.async_copy(b2_v.at[0], g2_hbm.at[pl.ds(base, C)], wsem)
                w1.wait()
                w2.wait()
        return 0

    lax.fori_loop(0, GPER // 2, pair, 0)


_sc_gather = functools.partial(
    pl.kernel,
    out_type=(jax.ShapeDtypeStruct((EHALF, D), jnp.float32),
              jax.ShapeDtypeStruct((EHALF, D), jnp.float32)),
    mesh=plsc.VectorSubcoreMesh(core_axis_name="c", subcore_axis_name="s",
                                num_cores=NC, num_subcores=NS),
    scratch_types=(pltpu.VMEM((GPER, C), jnp.int32),
                   pltpu.VMEM((GPER, C), jnp.int32),
                   pltpu.VMEM((2, C, D), jnp.float32),
                   pltpu.VMEM((2, C, D), jnp.float32),
                   pltpu.SemaphoreType.DMA,
                   pltpu.SemaphoreType.DMA),
)(_gather_body)


ROWS_PER_TILE = 1000


def _scatter_body(msg_hbm, dstp_hbm, zeros_hbm, agg_hbm,
                  accum_sh, idx_v, mbuf_v, lsem, asem):
    c = lax.axis_index("c")
    s = lax.axis_index("s")
    first = s * SPER

    @pl.when(s < N // ROWS_PER_TILE)
    def _():
        pltpu.sync_copy(zeros_hbm,
                        accum_sh.at[pl.ds(s * ROWS_PER_TILE, ROWS_PER_TILE)])

    pltpu.sync_copy(dstp_hbm.at[pl.ds(first, SPER)], idx_v)
    plsc.subcore_barrier()

    def valid(i):
        return jnp.logical_and(i < SPER, first + i < NCHUNK)

    def pair(i2, _):
        i = i2 * 2
        j = i + 1

        @pl.when(valid(i))
        def _():
            l1 = pltpu.async_copy(msg_hbm.at[c, pl.ds((first + i) * C, C)],
                                  mbuf_v.at[0], lsem)

            @pl.when(valid(j))
            def _():
                l2 = pltpu.async_copy(msg_hbm.at[c, pl.ds((first + j) * C, C)],
                                      mbuf_v.at[1], lsem)
                l1.wait()
                a1 = pltpu.async_copy(mbuf_v.at[0], accum_sh.at[idx_v.at[i]],
                                      asem, add=True)
                l2.wait()
                a2 = pltpu.async_copy(mbuf_v.at[1], accum_sh.at[idx_v.at[j]],
                                      asem, add=True)
                a1.wait()
                a2.wait()

            @pl.when(jnp.logical_not(valid(j)))
            def _():
                l1.wait()
                a1 = pltpu.async_copy(mbuf_v.at[0], accum_sh.at[idx_v.at[i]],
                                      asem, add=True)
                a1.wait()
        return 0

    lax.fori_loop(0, SPER // 2, pair, 0)
    plsc.subcore_barrier()

    @pl.when(s < N // ROWS_PER_TILE)
    def _():
        pltpu.sync_copy(accum_sh.at[pl.ds(s * ROWS_PER_TILE, ROWS_PER_TILE)],
                        agg_hbm.at[c, pl.ds(s * ROWS_PER_TILE, ROWS_PER_TILE)])


_sc_scatter = functools.partial(
    pl.kernel,
    out_type=jax.ShapeDtypeStruct((NC, N, D), jnp.float32),
    mesh=plsc.VectorSubcoreMesh(core_axis_name="c", subcore_axis_name="s",
                                num_cores=NC, num_subcores=NS),
    scratch_types=(pltpu.VMEM_SHARED((N, D), jnp.float32),
                   pltpu.VMEM((SPER, C), jnp.int32),
                   pltpu.VMEM((2, C, D), jnp.float32),
                   pltpu.SemaphoreType.DMA,
                   pltpu.SemaphoreType.DMA),
)(_scatter_body)


EB = 4000


def _edge_mlp_body(g1_ref, g2_ref, wp_ref, wq_ref, b1_ref, w2_ref, b2_ref,
                   out_ref):
    bf = jnp.bfloat16
    pre = (jnp.dot(g1_ref[...].astype(bf), wp_ref[...],
                   preferred_element_type=jnp.float32)
           + jnp.dot(g2_ref[...].astype(bf), wq_ref[...],
                     preferred_element_type=jnp.float32)
           + b1_ref[...])
    h1 = _elu(pre)
    msg = _elu(jnp.dot(h1.astype(bf), w2_ref[...],
                       preferred_element_type=jnp.float32)
               + b2_ref[...])
    out_ref[0] = msg[:, :D]
    out_ref[1] = msg[:, D:]


def _tc_edge(g1, g2, wp, wq, b1, w2, b2):
    return pl.pallas_call(
        _edge_mlp_body,
        grid=(EHALF // EB,),
        in_specs=[
            pl.BlockSpec((EB, D), lambda i: (i, 0)),
            pl.BlockSpec((EB, D), lambda i: (i, 0)),
            pl.BlockSpec((D, 2 * D), lambda i: (0, 0)),
            pl.BlockSpec((D, 2 * D), lambda i: (0, 0)),
            pl.BlockSpec((1, 2 * D), lambda i: (0, 0)),
            pl.BlockSpec((2 * D, 2 * D), lambda i: (0, 0)),
            pl.BlockSpec((1, 2 * D), lambda i: (0, 0)),
        ],
        out_specs=pl.BlockSpec((NC, EB, D), lambda i: (0, i, 0)),
        out_shape=jax.ShapeDtypeStruct((NC, EHALF, D), jnp.float32),
    )(g1, g2, wp, wq, b1, w2, b2)


RB = 2000


def _node_mlp_body(agg_ref, aggb_ref, m2w1_ref, m2b1_ref, m2w2_ref, m2b2_ref,
                   mow1_ref, mob1_ref, mow2_ref, mob2_ref,
                   wi1_ref, bi1_ref, h_ref):
    bf = jnp.bfloat16
    x = jnp.concatenate([agg_ref[0] + aggb_ref[0],
                         agg_ref[1] + aggb_ref[1]], axis=1)
    u = _elu(jnp.dot(x.astype(bf), m2w1_ref[...],
                     preferred_element_type=jnp.float32) + m2b1_ref[...])
    u = _elu(jnp.dot(u.astype(bf), m2w2_ref[...],
                     preferred_element_type=jnp.float32) + m2b2_ref[...])
    o = _elu(jnp.dot(u.astype(bf), mow1_ref[...],
                     preferred_element_type=jnp.float32) + mob1_ref[...])
    o = _elu(jnp.dot(o.astype(bf), mow2_ref[...],
                     preferred_element_type=jnp.float32) + mob2_ref[...])
    o = o[:, D:]
    h_ref[...] = jnp.maximum(
        jnp.dot(o.astype(bf), wi1_ref[...], preferred_element_type=jnp.float32)
        + bi1_ref[...], 0.0).astype(bf)


def _tc_node_mlp(agg, aggb, m2W1, m2b1, m2W2, m2b2, moW1, mob1, moW2, mob2,
                 Wi1, bi1):
    full = lambda r, c: pl.BlockSpec((r, c), lambda i: (0, 0))
    return pl.pallas_call(
        _node_mlp_body,
        grid=(N // RB,),
        in_specs=[
            pl.BlockSpec((NC, RB, D), lambda i: (0, i, 0)),
            pl.BlockSpec((NC, RB, D), lambda i: (0, i, 0)),
            full(2 * D, 2 * D), full(1, 2 * D),
            full(2 * D, 2 * D), full(1, 2 * D),
            full(2 * D, 2 * D), full(1, 2 * D),
            full(2 * D, 2 * D), full(1, 2 * D),
            full(D, 3 * D), full(1, 3 * D),
        ],
        out_specs=pl.BlockSpec((RB, 3 * D), lambda i: (i, 0)),
        out_shape=jax.ShapeDtypeStruct((N, 3 * D), jnp.bfloat16),
    )(agg, aggb, m2W1, m2b1.reshape(1, -1), m2W2, m2b2.reshape(1, -1),
      moW1, mob1.reshape(1, -1), moW2, mob2.reshape(1, -1),
      Wi1, bi1.reshape(1, -1))


FRB = 200


def _final_body(h_ref, w_ref, b_ref, out_ref):
    z = (jnp.dot(h_ref[...], w_ref[...], preferred_element_type=jnp.float32)
         + b_ref[...])
    out_ref[...] = 0.5 + 0.5 * jnp.tanh(0.5 * z)


def _tc_final(h, Wi2, bi2):
    return pl.pallas_call(
        _final_body,
        grid=(N // FRB,),
        in_specs=[
            pl.BlockSpec((FRB, 3 * D), lambda i: (i, 0)),
            pl.BlockSpec((3 * D, NUM_API), lambda i: (0, 0)),
            pl.BlockSpec((1, NUM_API), lambda i: (0, 0)),
        ],
        out_specs=pl.BlockSpec((FRB, NUM_API), lambda i: (i, 0)),
        out_shape=jax.ShapeDtypeStruct((N, NUM_API), jnp.float32),
    )(h, Wi2, bi2.reshape(1, -1))


def kernel(api_embeds, adjacency_matrix, edge_index,
           Wr1, br1, Wr2, br2,
           m1W1, m1b1, m1W2, m1b2,
           m2W1, m2b1, m2W2, m2b2,
           moW1, mob1, moW2, mob2,
           Wi1, bi1, Wi2, bi2):
    del adjacency_matrix, Wr1, br1, Wr2, br2
    src = edge_index[0]
    dst = edge_index[1]
    pad = ((0, NPAD - NCHUNK), (0, 0))
    dstp = [jnp.pad(dst[h * EHALF:(h + 1) * EHALF].reshape(NCHUNK, C), pad)
            for h in range(2)]
    srcp = [jnp.pad(src[h * EHALF:(h + 1) * EHALF].reshape(NCHUNK, C), pad)
            for h in range(2)]

    bf = jnp.bfloat16
    WP = (m1W1[:D] + m1W1[D:2 * D]).astype(bf)
    WQ = (m1W1[2 * D:3 * D] + m1W1[3 * D:]).astype(bf)
    b1r = m1b1.reshape(1, -1)
    W2 = m1W2.astype(bf)
    b2r = m1b2.reshape(1, -1)
    zeros = jnp.zeros((ROWS_PER_TILE, D), jnp.float32)

    g1a, g2a = _sc_gather(api_embeds, dstp[0], srcp[0])
    g1b, g2b = _sc_gather(api_embeds, dstp[1], srcp[1])
    msga = _tc_edge(g1a, g2a, WP, WQ, b1r, W2, b2r)
    agga = _sc_scatter(msga, dstp[0], zeros)
    msgb = _tc_edge(g1b, g2b, WP, WQ, b1r, W2, b2r)
    aggb = _sc_scatter(msgb, dstp[1], zeros)
    h = _tc_node_mlp(agga, aggb, m2W1.astype(bf), m2b1, m2W2.astype(bf), m2b2,
                     moW1.astype(bf), mob1, moW2.astype(bf), mob2,
                     Wi1.astype(bf), bi1)
    return _tc_final(h, Wi2.astype(bf), bi2)


# speedup vs baseline: 3.6324x; 1.0249x over previous
import functools

import jax
import jax.numpy as jnp
from jax import lax
from jax.experimental import pallas as pl
from jax.experimental.pallas import tpu as pltpu
from jax.experimental.pallas import tpu_sc as plsc

D = 128
N = 10000
NUM_API = 10000
E = 160000

NC = 2
NS = 16
C = 128
EHALF = E // 2
NCHUNK = EHALF // C
GC = 20
GPERL = 24
SPER = 40
NPAD = 768


def _elu(x):
    return jnp.where(x > 0, x, jnp.exp(jnp.minimum(x, 0.0)) - 1.0)


def _gather_body(a_hbm, dstp_hbm, srcp_hbm, g1_hbm, g2_hbm,
                 idxd_v, idxs_v, b1_v, b2_v, gsem, wsem):
    wid = lax.axis_index("s") * NC + lax.axis_index("c")
    first = wid * GC
    off = lax.rem(first, 8)
    start8 = pl.multiple_of(first - off, 8)

    pltpu.sync_copy(dstp_hbm.at[pl.ds(start8, GPERL)], idxd_v)
    pltpu.sync_copy(srcp_hbm.at[pl.ds(start8, GPERL)], idxs_v)

    def valid(i):
        return jnp.logical_and(i < GC, first + i < NCHUNK)

    def pair(i2, _):
        i = i2 * 2
        j = i + 1

        @pl.when(valid(i))
        def _():
            cp1 = pltpu.async_copy(a_hbm.at[idxd_v.at[off + i]], b1_v.at[0], gsem)
            cp2 = pltpu.async_copy(a_hbm.at[idxs_v.at[off + i]], b2_v.at[0], gsem)

            @pl.when(valid(j))
            def _():
                cp3 = pltpu.async_copy(a_hbm.at[idxd_v.at[off + j]], b1_v.at[1], gsem)
                cp4 = pltpu.async_copy(a_hbm.at[idxs_v.at[off + j]], b2_v.at[1], gsem)
                cp1.wait()
                cp2.wait()
                base = (first + i) * C
                w1 = pltpu.async_copy(b1_v.at[0], g1_hbm.at[pl.ds(base, C)], wsem)
                w2 = pltpu.async_copy(b2_v.at[0], g2_hbm.at[pl.ds(base, C)], wsem)
                cp3.wait()
                cp4.wait()
                basej = (first + j) * C
                w3 = pltpu.async_copy(b1_v.at[1], g1_hbm.at[pl.ds(basej, C)], wsem)
                w4 = pltpu.async_copy(b2_v.at[1], g2_hbm.at[pl.ds(basej, C)], wsem)
                w1.wait()
                w2.wait()
                w3.wait()
                w4.wait()

            @pl.when(jnp.logical_not(valid(j)))
            def _():
                cp1.wait()
                cp2.wait()
                base = (first + i) * C
                w1 = pltpu.async_copy(b1_v.at[0], g1_hbm.at[pl.ds(base, C)], wsem)
                w2 = pltpu.async_copy(b2_v.at[0], g2_hbm.at[pl.ds(base, C)], wsem)
                w1.wait()
                w2.wait()
        return 0

    lax.fori_loop(0, GC // 2, pair, 0)


_sc_gather = functools.partial(
    pl.kernel,
    out_type=(jax.ShapeDtypeStruct((EHALF, D), jnp.float32),
              jax.ShapeDtypeStruct((EHALF, D), jnp.float32)),
    mesh=plsc.VectorSubcoreMesh(core_axis_name="c", subcore_axis_name="s",
                                num_cores=NC, num_subcores=NS),
    scratch_types=(pltpu.VMEM((GPERL, C), jnp.int32),
                   pltpu.VMEM((GPERL, C), jnp.int32),
                   pltpu.VMEM((2, C, D), jnp.float32),
                   pltpu.VMEM((2, C, D), jnp.float32),
                   pltpu.SemaphoreType.DMA,
                   pltpu.SemaphoreType.DMA),
)(_gather_body)


ROWS_PER_TILE = 1000


def _scatter_body(msg_hbm, dstp_hbm, zeros_hbm, agg_hbm,
                  accum_sh, idx_v, mbuf_v, lsem, asem):
    c = lax.axis_index("c")
    s = lax.axis_index("s")
    first = s * SPER

    @pl.when(s < N // ROWS_PER_TILE)
    def _():
        pltpu.sync_copy(zeros_hbm,
                        accum_sh.at[pl.ds(s * ROWS_PER_TILE, ROWS_PER_TILE)])

    pltpu.sync_copy(dstp_hbm.at[pl.ds(first, SPER)], idx_v)
    plsc.subcore_barrier()

    def valid(i):
        return jnp.logical_and(i < SPER, first + i < NCHUNK)

    def pair(i2, _):
        i = i2 * 2
        j = i + 1

        @pl.when(valid(i))
        def _():
            l1 = pltpu.async_copy(msg_hbm.at[c, pl.ds((first + i) * C, C)],
                                  mbuf_v.at[0], lsem)

            @pl.when(valid(j))
            def _():
                l2 = pltpu.async_copy(msg_hbm.at[c, pl.ds((first + j) * C, C)],
                                      mbuf_v.at[1], lsem)
                l1.wait()
                a1 = pltpu.async_copy(mbuf_v.at[0], accum_sh.at[idx_v.at[i]],
                                      asem, add=True)
                l2.wait()
                a2 = pltpu.async_copy(mbuf_v.at[1], accum_sh.at[idx_v.at[j]],
                                      asem, add=True)
                a1.wait()
                a2.wait()

            @pl.when(jnp.logical_not(valid(j)))
            def _():
                l1.wait()
                a1 = pltpu.async_copy(mbuf_v.at[0], accum_sh.at[idx_v.at[i]],
                                      asem, add=True)
                a1.wait()
        return 0

    lax.fori_loop(0, SPER // 2, pair, 0)
    plsc.subcore_barrier()

    @pl.when(s < N // ROWS_PER_TILE)
    def _():
        pltpu.sync_copy(accum_sh.at[pl.ds(s * ROWS_PER_TILE, ROWS_PER_TILE)],
                        agg_hbm.at[c, pl.ds(s * ROWS_PER_TILE, ROWS_PER_TILE)])


_sc_scatter = functools.partial(
    pl.kernel,
    out_type=jax.ShapeDtypeStruct((NC, N, D), jnp.float32),
    mesh=plsc.VectorSubcoreMesh(core_axis_name="c", subcore_axis_name="s",
                                num_cores=NC, num_subcores=NS),
    scratch_types=(pltpu.VMEM_SHARED((N, D), jnp.float32),
                   pltpu.VMEM((SPER, C), jnp.int32),
                   pltpu.VMEM((2, C, D), jnp.float32),
                   pltpu.SemaphoreType.DMA,
                   pltpu.SemaphoreType.DMA),
)(_scatter_body)


EB = 4000


def _edge_mlp_body(g1_ref, g2_ref, wp_ref, wq_ref, b1_ref, w2_ref, b2_ref,
                   out_ref):
    bf = jnp.bfloat16
    pre = (jnp.dot(g1_ref[...].astype(bf), wp_ref[...],
                   preferred_element_type=jnp.float32)
           + jnp.dot(g2_ref[...].astype(bf), wq_ref[...],
                     preferred_element_type=jnp.float32)
           + b1_ref[...])
    h1 = _elu(pre)
    msg = _elu(jnp.dot(h1.astype(bf), w2_ref[...],
                       preferred_element_type=jnp.float32)
               + b2_ref[...])
    out_ref[0] = msg[:, :D]
    out_ref[1] = msg[:, D:]


def _tc_edge(g1, g2, wp, wq, b1, w2, b2):
    return pl.pallas_call(
        _edge_mlp_body,
        grid=(EHALF // EB,),
        in_specs=[
            pl.BlockSpec((EB, D), lambda i: (i, 0)),
            pl.BlockSpec((EB, D), lambda i: (i, 0)),
            pl.BlockSpec((D, 2 * D), lambda i: (0, 0)),
            pl.BlockSpec((D, 2 * D), lambda i: (0, 0)),
            pl.BlockSpec((1, 2 * D), lambda i: (0, 0)),
            pl.BlockSpec((2 * D, 2 * D), lambda i: (0, 0)),
            pl.BlockSpec((1, 2 * D), lambda i: (0, 0)),
        ],
        out_specs=pl.BlockSpec((NC, EB, D), lambda i: (0, i, 0)),
        out_shape=jax.ShapeDtypeStruct((NC, EHALF, D), jnp.float32),
    )(g1, g2, wp, wq, b1, w2, b2)


RB = 2000


def _node_mlp_body(agg_ref, aggb_ref, m2w1_ref, m2b1_ref, m2w2_ref, m2b2_ref,
                   mow1_ref, mob1_ref, mow2_ref, mob2_ref,
                   wi1_ref, bi1_ref, h_ref):
    bf = jnp.bfloat16
    x = jnp.concatenate([agg_ref[0] + aggb_ref[0],
                         agg_ref[1] + aggb_ref[1]], axis=1)
    u = _elu(jnp.dot(x.astype(bf), m2w1_ref[...],
                     preferred_element_type=jnp.float32) + m2b1_ref[...])
    u = _elu(jnp.dot(u.astype(bf), m2w2_ref[...],
                     preferred_element_type=jnp.float32) + m2b2_ref[...])
    o = _elu(jnp.dot(u.astype(bf), mow1_ref[...],
                     preferred_element_type=jnp.float32) + mob1_ref[...])
    o = _elu(jnp.dot(o.astype(bf), mow2_ref[...],
                     preferred_element_type=jnp.float32) + mob2_ref[...])
    o = o[:, D:]
    h_ref[...] = jnp.maximum(
        jnp.dot(o.astype(bf), wi1_ref[...], preferred_element_type=jnp.float32)
        + bi1_ref[...], 0.0).astype(bf)


def _tc_node_mlp(agg, aggb, m2W1, m2b1, m2W2, m2b2, moW1, mob1, moW2, mob2,
                 Wi1, bi1):
    full = lambda r, c: pl.BlockSpec((r, c), lambda i: (0, 0))
    return pl.pallas_call(
        _node_mlp_body,
        grid=(N // RB,),
        in_specs=[
            pl.BlockSpec((NC, RB, D), lambda i: (0, i, 0)),
            pl.BlockSpec((NC, RB, D), lambda i: (0, i, 0)),
            full(2 * D, 2 * D), full(1, 2 * D),
            full(2 * D, 2 * D), full(1, 2 * D),
            full(2 * D, 2 * D), full(1, 2 * D),
            full(2 * D, 2 * D), full(1, 2 * D),
            full(D, 3 * D), full(1, 3 * D),
        ],
        out_specs=pl.BlockSpec((RB, 3 * D), lambda i: (i, 0)),
        out_shape=jax.ShapeDtypeStruct((N, 3 * D), jnp.bfloat16),
    )(agg, aggb, m2W1, m2b1.reshape(1, -1), m2W2, m2b2.reshape(1, -1),
      moW1, mob1.reshape(1, -1), moW2, mob2.reshape(1, -1),
      Wi1, bi1.reshape(1, -1))


FRB = 400


def _final_body(h_ref, w_ref, b_ref, out_ref):
    z = (jnp.dot(h_ref[...], w_ref[...], preferred_element_type=jnp.float32)
         + b_ref[...])
    out_ref[...] = 0.5 + 0.5 * jnp.tanh(0.5 * z)


def _tc_final(h, Wi2, bi2):
    return pl.pallas_call(
        _final_body,
        grid=(N // FRB,),
        in_specs=[
            pl.BlockSpec((FRB, 3 * D), lambda i: (i, 0)),
            pl.BlockSpec((3 * D, NUM_API), lambda i: (0, 0)),
            pl.BlockSpec((1, NUM_API), lambda i: (0, 0)),
        ],
        out_specs=pl.BlockSpec((FRB, NUM_API), lambda i: (i, 0)),
        out_shape=jax.ShapeDtypeStruct((N, NUM_API), jnp.float32),
    )(h, Wi2, bi2.reshape(1, -1))


def kernel(api_embeds, adjacency_matrix, edge_index,
           Wr1, br1, Wr2, br2,
           m1W1, m1b1, m1W2, m1b2,
           m2W1, m2b1, m2W2, m2b2,
           moW1, mob1, moW2, mob2,
           Wi1, bi1, Wi2, bi2):
    del adjacency_matrix, Wr1, br1, Wr2, br2
    src = edge_index[0]
    dst = edge_index[1]
    pad = ((0, NPAD - NCHUNK), (0, 0))
    dstp = [jnp.pad(dst[h * EHALF:(h + 1) * EHALF].reshape(NCHUNK, C), pad)
            for h in range(2)]
    srcp = [jnp.pad(src[h * EHALF:(h + 1) * EHALF].reshape(NCHUNK, C), pad)
            for h in range(2)]

    bf = jnp.bfloat16
    WP = (m1W1[:D] + m1W1[D:2 * D]).astype(bf)
    WQ = (m1W1[2 * D:3 * D] + m1W1[3 * D:]).astype(bf)
    b1r = m1b1.reshape(1, -1)
    W2 = m1W2.astype(bf)
    b2r = m1b2.reshape(1, -1)
    zeros = jnp.zeros((ROWS_PER_TILE, D), jnp.float32)

    g1a, g2a = _sc_gather(api_embeds, dstp[0], srcp[0])
    g1b, g2b = _sc_gather(api_embeds, dstp[1], srcp[1])
    msga = _tc_edge(g1a, g2a, WP, WQ, b1r, W2, b2r)
    agga = _sc_scatter(msga, dstp[0], zeros)
    msgb = _tc_edge(g1b, g2b, WP, WQ, b1r, W2, b2r)
    aggb = _sc_scatter(msgb, dstp[1], zeros)
    h = _tc_node_mlp(agga, aggb, m2W1.astype(bf), m2b1, m2W2.astype(bf), m2b2,
                     moW1.astype(bf), mob1, moW2.astype(bf), mob2,
                     Wi1.astype(bf), bi1)
    return _tc_final(h, Wi2.astype(bf), bi2)


# speedup vs baseline: 3.6850x; 1.0145x over previous
import functools

import jax
import jax.numpy as jnp
from jax import lax
from jax.experimental import pallas as pl
from jax.experimental.pallas import tpu as pltpu
from jax.experimental.pallas import tpu_sc as plsc

D = 128
N = 10000
NUM_API = 10000
E = 160000

NC = 2
NS = 16
C = 128
EHALF = E // 2
NCHUNK = EHALF // C
GC = 20
GPERL = 24
SPER = 40
NPAD = 768


def _elu(x):
    return jnp.where(x > 0, x, jnp.exp(jnp.minimum(x, 0.0)) - 1.0)


def _gather_body(a_hbm, dstp_hbm, srcp_hbm, g1_hbm, g2_hbm,
                 idxd_v, idxs_v, b1_v, b2_v, gsem, wsem):
    wid = lax.axis_index("s") * NC + lax.axis_index("c")
    first = wid * GC
    off = lax.rem(first, 8)
    start8 = pl.multiple_of(first - off, 8)

    pltpu.sync_copy(dstp_hbm.at[pl.ds(start8, GPERL)], idxd_v)
    pltpu.sync_copy(srcp_hbm.at[pl.ds(start8, GPERL)], idxs_v)

    def valid(i):
        return jnp.logical_and(i < GC, first + i < NCHUNK)

    def pair(i2, _):
        i = i2 * 2
        j = i + 1

        @pl.when(valid(i))
        def _():
            cp1 = pltpu.async_copy(a_hbm.at[idxd_v.at[off + i]], b1_v.at[0], gsem)
            cp2 = pltpu.async_copy(a_hbm.at[idxs_v.at[off + i]], b2_v.at[0], gsem)

            @pl.when(valid(j))
            def _():
                cp3 = pltpu.async_copy(a_hbm.at[idxd_v.at[off + j]], b1_v.at[1], gsem)
                cp4 = pltpu.async_copy(a_hbm.at[idxs_v.at[off + j]], b2_v.at[1], gsem)
                cp1.wait()
                cp2.wait()
                base = (first + i) * C
                w1 = pltpu.async_copy(b1_v.at[0], g1_hbm.at[pl.ds(base, C)], wsem)
                w2 = pltpu.async_copy(b2_v.at[0], g2_hbm.at[pl.ds(base, C)], wsem)
                cp3.wait()
                cp4.wait()
                basej = (first + j) * C
                w3 = pltpu.async_copy(b1_v.at[1], g1_hbm.at[pl.ds(basej, C)], wsem)
                w4 = pltpu.async_copy(b2_v.at[1], g2_hbm.at[pl.ds(basej, C)], wsem)
                w1.wait()
                w2.wait()
                w3.wait()
                w4.wait()

            @pl.when(jnp.logical_not(valid(j)))
            def _():
                cp1.wait()
                cp2.wait()
                base = (first + i) * C
                w1 = pltpu.async_copy(b1_v.at[0], g1_hbm.at[pl.ds(base, C)], wsem)
                w2 = pltpu.async_copy(b2_v.at[0], g2_hbm.at[pl.ds(base, C)], wsem)
                w1.wait()
                w2.wait()
        return 0

    lax.fori_loop(0, GC // 2, pair, 0)


_sc_gather = functools.partial(
    pl.kernel,
    out_type=(jax.ShapeDtypeStruct((EHALF, D), jnp.float32),
              jax.ShapeDtypeStruct((EHALF, D), jnp.float32)),
    mesh=plsc.VectorSubcoreMesh(core_axis_name="c", subcore_axis_name="s",
                                num_cores=NC, num_subcores=NS),
    scratch_types=(pltpu.VMEM((GPERL, C), jnp.int32),
                   pltpu.VMEM((GPERL, C), jnp.int32),
                   pltpu.VMEM((2, C, D), jnp.float32),
                   pltpu.VMEM((2, C, D), jnp.float32),
                   pltpu.SemaphoreType.DMA,
                   pltpu.SemaphoreType.DMA),
)(_gather_body)


ROWS_PER_TILE = 1000


def _scatter_body(msg_hbm, dstp_hbm, zeros_hbm, agg_hbm,
                  accum_sh, idx_v, mbuf_v, lsem, asem):
    c = lax.axis_index("c")
    s = lax.axis_index("s")
    first = s * SPER

    @pl.when(s < N // ROWS_PER_TILE)
    def _():
        pltpu.sync_copy(zeros_hbm,
                        accum_sh.at[pl.ds(s * ROWS_PER_TILE, ROWS_PER_TILE)])

    pltpu.sync_copy(dstp_hbm.at[pl.ds(first, SPER)], idx_v)
    plsc.subcore_barrier()

    def valid(i):
        return jnp.logical_and(i < SPER, first + i < NCHUNK)

    def pair(i2, _):
        i = i2 * 2
        j = i + 1

        @pl.when(valid(i))
        def _():
            l1 = pltpu.async_copy(msg_hbm.at[c, pl.ds((first + i) * C, C)],
                                  mbuf_v.at[0], lsem)

            @pl.when(valid(j))
            def _():
                l2 = pltpu.async_copy(msg_hbm.at[c, pl.ds((first + j) * C, C)],
                                      mbuf_v.at[1], lsem)
                l1.wait()
                a1 = pltpu.async_copy(mbuf_v.at[0], accum_sh.at[idx_v.at[i]],
                                      asem, add=True)
                l2.wait()
                a2 = pltpu.async_copy(mbuf_v.at[1], accum_sh.at[idx_v.at[j]],
                                      asem, add=True)
                a1.wait()
                a2.wait()

            @pl.when(jnp.logical_not(valid(j)))
            def _():
                l1.wait()
                a1 = pltpu.async_copy(mbuf_v.at[0], accum_sh.at[idx_v.at[i]],
                                      asem, add=True)
                a1.wait()
        return 0

    lax.fori_loop(0, SPER // 2, pair, 0)
    plsc.subcore_barrier()

    @pl.when(s < N // ROWS_PER_TILE)
    def _():
        pltpu.sync_copy(accum_sh.at[pl.ds(s * ROWS_PER_TILE, ROWS_PER_TILE)],
                        agg_hbm.at[c, pl.ds(s * ROWS_PER_TILE, ROWS_PER_TILE)])


_sc_scatter = functools.partial(
    pl.kernel,
    out_type=jax.ShapeDtypeStruct((NC, N, D), jnp.float32),
    mesh=plsc.VectorSubcoreMesh(core_axis_name="c", subcore_axis_name="s",
                                num_cores=NC, num_subcores=NS),
    scratch_types=(pltpu.VMEM_SHARED((N, D), jnp.float32),
                   pltpu.VMEM((SPER, C), jnp.int32),
                   pltpu.VMEM((2, C, D), jnp.float32),
                   pltpu.SemaphoreType.DMA,
                   pltpu.SemaphoreType.DMA),
)(_scatter_body)


EB = 4000


def _edge_mlp_body(g1_ref, g2_ref, wp_ref, wq_ref, b1_ref, w2_ref, b2_ref,
                   out_ref):
    bf = jnp.bfloat16
    pre = (jnp.dot(g1_ref[...].astype(bf), wp_ref[...],
                   preferred_element_type=jnp.float32)
           + jnp.dot(g2_ref[...].astype(bf), wq_ref[...],
                     preferred_element_type=jnp.float32)
           + b1_ref[...])
    h1 = _elu(pre)
    msg = _elu(jnp.dot(h1.astype(bf), w2_ref[...],
                       preferred_element_type=jnp.float32)
               + b2_ref[...])
    out_ref[0] = msg[:, :D]
    out_ref[1] = msg[:, D:]


def _tc_edge(g1, g2, wp, wq, b1, w2, b2):
    return pl.pallas_call(
        _edge_mlp_body,
        grid=(EHALF // EB,),
        in_specs=[
            pl.BlockSpec((EB, D), lambda i: (i, 0)),
            pl.BlockSpec((EB, D), lambda i: (i, 0)),
            pl.BlockSpec((D, 2 * D), lambda i: (0, 0)),
            pl.BlockSpec((D, 2 * D), lambda i: (0, 0)),
            pl.BlockSpec((1, 2 * D), lambda i: (0, 0)),
            pl.BlockSpec((2 * D, 2 * D), lambda i: (0, 0)),
            pl.BlockSpec((1, 2 * D), lambda i: (0, 0)),
        ],
        out_specs=pl.BlockSpec((NC, EB, D), lambda i: (0, i, 0)),
        out_shape=jax.ShapeDtypeStruct((NC, EHALF, D), jnp.float32),
    )(g1, g2, wp, wq, b1, w2, b2)


FRB = 400


def _final_body(agg_ref, aggb_ref, m2w1_ref, m2b1_ref, m2w2_ref, m2b2_ref,
                mow1_ref, mob1_ref, mow2_ref, mob2_ref,
                wi1_ref, bi1_ref, w_ref, b_ref, out_ref):
    bf = jnp.bfloat16
    x = jnp.concatenate([agg_ref[0] + aggb_ref[0],
                         agg_ref[1] + aggb_ref[1]], axis=1)
    u = _elu(jnp.dot(x.astype(bf), m2w1_ref[...],
                     preferred_element_type=jnp.float32) + m2b1_ref[...])
    u = _elu(jnp.dot(u.astype(bf), m2w2_ref[...],
                     preferred_element_type=jnp.float32) + m2b2_ref[...])
    o = _elu(jnp.dot(u.astype(bf), mow1_ref[...],
                     preferred_element_type=jnp.float32) + mob1_ref[...])
    o = _elu(jnp.dot(o.astype(bf), mow2_ref[...],
                     preferred_element_type=jnp.float32) + mob2_ref[...])
    o = o[:, D:]
    h = jnp.maximum(
        jnp.dot(o.astype(bf), wi1_ref[...], preferred_element_type=jnp.float32)
        + bi1_ref[...], 0.0)
    z = (jnp.dot(h.astype(bf), w_ref[...], preferred_element_type=jnp.float32)
         + b_ref[...])
    out_ref[...] = 0.5 + 0.5 * jnp.tanh(0.5 * z)


def _tc_final(agga, aggb, m2W1, m2b1, m2W2, m2b2, moW1, mob1, moW2, mob2,
              Wi1, bi1, Wi2, bi2):
    full = lambda r, c: pl.BlockSpec((r, c), lambda i: (0, 0))
    return pl.pallas_call(
        _final_body,
        grid=(N // FRB,),
        in_specs=[
            pl.BlockSpec((NC, FRB, D), lambda i: (0, i, 0)),
            pl.BlockSpec((NC, FRB, D), lambda i: (0, i, 0)),
            full(2 * D, 2 * D), full(1, 2 * D),
            full(2 * D, 2 * D), full(1, 2 * D),
            full(2 * D, 2 * D), full(1, 2 * D),
            full(2 * D, 2 * D), full(1, 2 * D),
            full(D, 3 * D), full(1, 3 * D),
            full(3 * D, NUM_API), full(1, NUM_API),
        ],
        out_specs=pl.BlockSpec((FRB, NUM_API), lambda i: (i, 0)),
        out_shape=jax.ShapeDtypeStruct((N, NUM_API), jnp.float32),
    )(agga, aggb, m2W1, m2b1.reshape(1, -1), m2W2, m2b2.reshape(1, -1),
      moW1, mob1.reshape(1, -1), moW2, mob2.reshape(1, -1),
      Wi1, bi1.reshape(1, -1), Wi2, bi2.reshape(1, -1))


def kernel(api_embeds, adjacency_matrix, edge_index,
           Wr1, br1, Wr2, br2,
           m1W1, m1b1, m1W2, m1b2,
           m2W1, m2b1, m2W2, m2b2,
           moW1, mob1, moW2, mob2,
           Wi1, bi1, Wi2, bi2):
    del adjacency_matrix, Wr1, br1, Wr2, br2
    src = edge_index[0]
    dst = edge_index[1]
    pad = ((0, NPAD - NCHUNK), (0, 0))
    dstp = [jnp.pad(dst[h * EHALF:(h + 1) * EHALF].reshape(NCHUNK, C), pad)
            for h in range(2)]
    srcp = [jnp.pad(src[h * EHALF:(h + 1) * EHALF].reshape(NCHUNK, C), pad)
            for h in range(2)]

    bf = jnp.bfloat16
    WP = (m1W1[:D] + m1W1[D:2 * D]).astype(bf)
    WQ = (m1W1[2 * D:3 * D] + m1W1[3 * D:]).astype(bf)
    b1r = m1b1.reshape(1, -1)
    W2 = m1W2.astype(bf)
    b2r = m1b2.reshape(1, -1)
    zeros = jnp.zeros((ROWS_PER_TILE, D), jnp.float32)

    g1a, g2a = _sc_gather(api_embeds, dstp[0], srcp[0])
    g1b, g2b = _sc_gather(api_embeds, dstp[1], srcp[1])
    msga = _tc_edge(g1a, g2a, WP, WQ, b1r, W2, b2r)
    agga = _sc_scatter(msga, dstp[0], zeros)
    msgb = _tc_edge(g1b, g2b, WP, WQ, b1r, W2, b2r)
    aggb = _sc_scatter(msgb, dstp[1], zeros)
    return _tc_final(agga, aggb, m2W1.astype(bf), m2b1, m2W2.astype(bf), m2b2,
                     moW1.astype(bf), mob1, moW2.astype(bf), mob2,
                     Wi1.astype(bf), bi1, Wi2.astype(bf), bi2)


# speedup vs baseline: 3.7057x; 1.0056x over previous
import functools

import jax
import jax.numpy as jnp
from jax import lax
from jax.experimental import pallas as pl
from jax.experimental.pallas import tpu as pltpu
from jax.experimental.pallas import tpu_sc as plsc

D = 128
N = 10000
NUM_API = 10000
E = 160000

NC = 2
NS = 16
C = 128
EHALF = E // 2
NCHUNK = EHALF // C
GC = 20
GPERL = 24
SPER = 40
NPAD = 768


def _elu(x):
    return jnp.where(x > 0, x, jnp.exp(jnp.minimum(x, 0.0)) - 1.0)


def _gather_body(a_hbm, dstp_hbm, srcp_hbm, g1_hbm, g2_hbm,
                 idxd_v, idxs_v, b1_v, b2_v, gsem, wsem):
    wid = lax.axis_index("s") * NC + lax.axis_index("c")
    first = wid * GC
    off = lax.rem(first, 8)
    start8 = pl.multiple_of(first - off, 8)

    pltpu.sync_copy(dstp_hbm.at[pl.ds(start8, GPERL)], idxd_v)
    pltpu.sync_copy(srcp_hbm.at[pl.ds(start8, GPERL)], idxs_v)

    def valid(i):
        return jnp.logical_and(i < GC, first + i < NCHUNK)

    def pair(i2, _):
        i = i2 * 2
        j = i + 1

        @pl.when(valid(i))
        def _():
            cp1 = pltpu.async_copy(a_hbm.at[idxd_v.at[off + i]], b1_v.at[0], gsem)
            cp2 = pltpu.async_copy(a_hbm.at[idxs_v.at[off + i]], b2_v.at[0], gsem)

            @pl.when(valid(j))
            def _():
                cp3 = pltpu.async_copy(a_hbm.at[idxd_v.at[off + j]], b1_v.at[1], gsem)
                cp4 = pltpu.async_copy(a_hbm.at[idxs_v.at[off + j]], b2_v.at[1], gsem)
                cp1.wait()
                cp2.wait()
                base = (first + i) * C
                w1 = pltpu.async_copy(b1_v.at[0], g1_hbm.at[pl.ds(base, C)], wsem)
                w2 = pltpu.async_copy(b2_v.at[0], g2_hbm.at[pl.ds(base, C)], wsem)
                cp3.wait()
                cp4.wait()
                basej = (first + j) * C
                w3 = pltpu.async_copy(b1_v.at[1], g1_hbm.at[pl.ds(basej, C)], wsem)
                w4 = pltpu.async_copy(b2_v.at[1], g2_hbm.at[pl.ds(basej, C)], wsem)
                w1.wait()
                w2.wait()
                w3.wait()
                w4.wait()

            @pl.when(jnp.logical_not(valid(j)))
            def _():
                cp1.wait()
                cp2.wait()
                base = (first + i) * C
                w1 = pltpu.async_copy(b1_v.at[0], g1_hbm.at[pl.ds(base, C)], wsem)
                w2 = pltpu.async_copy(b2_v.at[0], g2_hbm.at[pl.ds(base, C)], wsem)
                w1.wait()
                w2.wait()
        return 0

    lax.fori_loop(0, GC // 2, pair, 0)


_sc_gather = functools.partial(
    pl.kernel,
    out_type=(jax.ShapeDtypeStruct((EHALF, D), jnp.float32),
              jax.ShapeDtypeStruct((EHALF, D), jnp.float32)),
    mesh=plsc.VectorSubcoreMesh(core_axis_name="c", subcore_axis_name="s",
                                num_cores=NC, num_subcores=NS),
    scratch_types=(pltpu.VMEM((GPERL, C), jnp.int32),
                   pltpu.VMEM((GPERL, C), jnp.int32),
                   pltpu.VMEM((2, C, D), jnp.float32),
                   pltpu.VMEM((2, C, D), jnp.float32),
                   pltpu.SemaphoreType.DMA,
                   pltpu.SemaphoreType.DMA),
)(_gather_body)


ROWS_PER_TILE = 1000


def _scatter_body(msg_hbm, dstp_hbm, zeros_hbm, agg_hbm,
                  accum_sh, idx_v, mbuf_v, lsem, asem):
    c = lax.axis_index("c")
    s = lax.axis_index("s")
    first = s * SPER

    @pl.when(s < N // ROWS_PER_TILE)
    def _():
        pltpu.sync_copy(zeros_hbm,
                        accum_sh.at[pl.ds(s * ROWS_PER_TILE, ROWS_PER_TILE)])

    pltpu.sync_copy(dstp_hbm.at[pl.ds(first, SPER)], idx_v)
    plsc.subcore_barrier()

    def valid(i):
        return jnp.logical_and(i < SPER, first + i < NCHUNK)

    def pair(i2, _):
        i = i2 * 2
        j = i + 1

        @pl.when(valid(i))
        def _():
            l1 = pltpu.async_copy(msg_hbm.at[c, pl.ds((first + i) * C, C)],
                                  mbuf_v.at[0], lsem)

            @pl.when(valid(j))
            def _():
                l2 = pltpu.async_copy(msg_hbm.at[c, pl.ds((first + j) * C, C)],
                                      mbuf_v.at[1], lsem)
                l1.wait()
                a1 = pltpu.async_copy(mbuf_v.at[0], accum_sh.at[idx_v.at[i]],
                                      asem, add=True)
                l2.wait()
                a2 = pltpu.async_copy(mbuf_v.at[1], accum_sh.at[idx_v.at[j]],
                                      asem, add=True)
                a1.wait()
                a2.wait()

            @pl.when(jnp.logical_not(valid(j)))
            def _():
                l1.wait()
                a1 = pltpu.async_copy(mbuf_v.at[0], accum_sh.at[idx_v.at[i]],
                                      asem, add=True)
                a1.wait()
        return 0

    lax.fori_loop(0, SPER // 2, pair, 0)
    plsc.subcore_barrier()

    @pl.when(s < N // ROWS_PER_TILE)
    def _():
        pltpu.sync_copy(accum_sh.at[pl.ds(s * ROWS_PER_TILE, ROWS_PER_TILE)],
                        agg_hbm.at[c, pl.ds(s * ROWS_PER_TILE, ROWS_PER_TILE)])


_sc_scatter = functools.partial(
    pl.kernel,
    out_type=jax.ShapeDtypeStruct((NC, N, D), jnp.float32),
    mesh=plsc.VectorSubcoreMesh(core_axis_name="c", subcore_axis_name="s",
                                num_cores=NC, num_subcores=NS),
    scratch_types=(pltpu.VMEM_SHARED((N, D), jnp.float32),
                   pltpu.VMEM((SPER, C), jnp.int32),
                   pltpu.VMEM((2, C, D), jnp.float32),
                   pltpu.SemaphoreType.DMA,
                   pltpu.SemaphoreType.DMA),
)(_scatter_body)


EB = 8000


def _edge_mlp_body(g1_ref, g2_ref, wp_ref, wq_ref, b1_ref, w2_ref, b2_ref,
                   out_ref):
    bf = jnp.bfloat16
    pre = (jnp.dot(g1_ref[...].astype(bf), wp_ref[...],
                   preferred_element_type=jnp.float32)
           + jnp.dot(g2_ref[...].astype(bf), wq_ref[...],
                     preferred_element_type=jnp.float32)
           + b1_ref[...])
    h1 = _elu(pre)
    msg = _elu(jnp.dot(h1.astype(bf), w2_ref[...],
                       preferred_element_type=jnp.float32)
               + b2_ref[...])
    out_ref[0] = msg[:, :D]
    out_ref[1] = msg[:, D:]


def _tc_edge(g1, g2, wp, wq, b1, w2, b2):
    return pl.pallas_call(
        _edge_mlp_body,
        grid=(EHALF // EB,),
        in_specs=[
            pl.BlockSpec((EB, D), lambda i: (i, 0)),
            pl.BlockSpec((EB, D), lambda i: (i, 0)),
            pl.BlockSpec((D, 2 * D), lambda i: (0, 0)),
            pl.BlockSpec((D, 2 * D), lambda i: (0, 0)),
            pl.BlockSpec((1, 2 * D), lambda i: (0, 0)),
            pl.BlockSpec((2 * D, 2 * D), lambda i: (0, 0)),
            pl.BlockSpec((1, 2 * D), lambda i: (0, 0)),
        ],
        out_specs=pl.BlockSpec((NC, EB, D), lambda i: (0, i, 0)),
        out_shape=jax.ShapeDtypeStruct((NC, EHALF, D), jnp.float32),
    )(g1, g2, wp, wq, b1, w2, b2)


FRB = 400


def _final_body(agg_ref, aggb_ref, m2w1_ref, m2b1_ref, m2w2_ref, m2b2_ref,
                mow1_ref, mob1_ref, mow2_ref, mob2_ref,
                wi1_ref, bi1_ref, w_ref, b_ref, out_ref):
    bf = jnp.bfloat16
    x = jnp.concatenate([agg_ref[0] + aggb_ref[0],
                         agg_ref[1] + aggb_ref[1]], axis=1)
    u = _elu(jnp.dot(x.astype(bf), m2w1_ref[...],
                     preferred_element_type=jnp.float32) + m2b1_ref[...])
    u = _elu(jnp.dot(u.astype(bf), m2w2_ref[...],
                     preferred_element_type=jnp.float32) + m2b2_ref[...])
    o = _elu(jnp.dot(u.astype(bf), mow1_ref[...],
                     preferred_element_type=jnp.float32) + mob1_ref[...])
    o = _elu(jnp.dot(o.astype(bf), mow2_ref[...],
                     preferred_element_type=jnp.float32) + mob2_ref[...])
    o = o[:, D:]
    h = jnp.maximum(
        jnp.dot(o.astype(bf), wi1_ref[...], preferred_element_type=jnp.float32)
        + bi1_ref[...], 0.0)
    z = (jnp.dot(h.astype(bf), w_ref[...], preferred_element_type=jnp.float32)
         + b_ref[...])
    out_ref[...] = 0.5 + 0.5 * jnp.tanh(0.5 * z)


def _tc_final(agga, aggb, m2W1, m2b1, m2W2, m2b2, moW1, mob1, moW2, mob2,
              Wi1, bi1, Wi2, bi2):
    full = lambda r, c: pl.BlockSpec((r, c), lambda i: (0, 0))
    return pl.pallas_call(
        _final_body,
        grid=(N // FRB,),
        in_specs=[
            pl.BlockSpec((NC, FRB, D), lambda i: (0, i, 0)),
            pl.BlockSpec((NC, FRB, D), lambda i: (0, i, 0)),
            full(2 * D, 2 * D), full(1, 2 * D),
            full(2 * D, 2 * D), full(1, 2 * D),
            full(2 * D, 2 * D), full(1, 2 * D),
            full(2 * D, 2 * D), full(1, 2 * D),
            full(D, 3 * D), full(1, 3 * D),
            full(3 * D, NUM_API), full(1, NUM_API),
        ],
        out_specs=pl.BlockSpec((FRB, NUM_API), lambda i: (i, 0)),
        out_shape=jax.ShapeDtypeStruct((N, NUM_API), jnp.float32),
    )(agga, aggb, m2W1, m2b1.reshape(1, -1), m2W2, m2b2.reshape(1, -1),
      moW1, mob1.reshape(1, -1), moW2, mob2.reshape(1, -1),
      Wi1, bi1.reshape(1, -1), Wi2, bi2.reshape(1, -1))


def kernel(api_embeds, adjacency_matrix, edge_index,
           Wr1, br1, Wr2, br2,
           m1W1, m1b1, m1W2, m1b2,
           m2W1, m2b1, m2W2, m2b2,
           moW1, mob1, moW2, mob2,
           Wi1, bi1, Wi2, bi2):
    del adjacency_matrix, Wr1, br1, Wr2, br2
    src = edge_index[0]
    dst = edge_index[1]
    pad = ((0, NPAD - NCHUNK), (0, 0))
    dstp = [jnp.pad(dst[h * EHALF:(h + 1) * EHALF].reshape(NCHUNK, C), pad)
            for h in range(2)]
    srcp = [jnp.pad(src[h * EHALF:(h + 1) * EHALF].reshape(NCHUNK, C), pad)
            for h in range(2)]

    bf = jnp.bfloat16
    WP = (m1W1[:D] + m1W1[D:2 * D]).astype(bf)
    WQ = (m1W1[2 * D:3 * D] + m1W1[3 * D:]).astype(bf)
    b1r = m1b1.reshape(1, -1)
    W2 = m1W2.astype(bf)
    b2r = m1b2.reshape(1, -1)
    zeros = jnp.zeros((ROWS_PER_TILE, D), jnp.float32)

    g1a, g2a = _sc_gather(api_embeds, dstp[0], srcp[0])
    g1b, g2b = _sc_gather(api_embeds, dstp[1], srcp[1])
    msga = _tc_edge(g1a, g2a, WP, WQ, b1r, W2, b2r)
    agga = _sc_scatter(msga, dstp[0], zeros)
    msgb = _tc_edge(g1b, g2b, WP, WQ, b1r, W2, b2r)
    aggb = _sc_scatter(msgb, dstp[1], zeros)
    return _tc_final(agga, aggb, m2W1.astype(bf), m2b1, m2W2.astype(bf), m2b2,
                     moW1.astype(bf), mob1, moW2.astype(bf), mob2,
                     Wi1.astype(bf), bi1, Wi2.astype(bf), bi2)


# speedup vs baseline: 3.9936x; 1.0777x over previous
import functools

import jax
import jax.numpy as jnp
from jax import lax
from jax.experimental import pallas as pl
from jax.experimental.pallas import tpu as pltpu
from jax.experimental.pallas import tpu_sc as plsc

D = 128
N = 10000
NUM_API = 10000
E = 160000

NC = 2
NS = 16
EHALF = E // 2

GCH = 80
GNCHUNK = EHALF // GCH
GC = 32
GNPAD = 1024

C = 128
NCHUNK = EHALF // C
SPER = 40
NPAD = 768


def _elu(x):
    return jnp.where(x > 0, x, jnp.exp(jnp.minimum(x, 0.0)) - 1.0)


def _gather_body(a_hbm, dstp_hbm, srcp_hbm, g1_hbm, g2_hbm,
                 table_sh, idxd_v, idxs_v, b1_v, b2_v, gsem, wsem):
    wid = lax.axis_index("s") * NC + lax.axis_index("c")
    s = lax.axis_index("s")
    first = wid * GC

    @pl.when(s < 10)
    def _():
        pltpu.sync_copy(a_hbm.at[pl.ds(s * 1000, 1000)],
                        table_sh.at[pl.ds(s * 1000, 1000)])

    pltpu.sync_copy(dstp_hbm.at[pl.ds(first, GC)], idxd_v)
    pltpu.sync_copy(srcp_hbm.at[pl.ds(first, GC)], idxs_v)
    plsc.subcore_barrier()

    def valid(i):
        return jnp.logical_and(i < GC, first + i < GNCHUNK)

    def pair(i2, _):
        i = i2 * 2
        j = i + 1

        @pl.when(valid(i))
        def _():
            cp1 = pltpu.async_copy(table_sh.at[idxd_v.at[i]], b1_v.at[0], gsem)
            cp2 = pltpu.async_copy(table_sh.at[idxs_v.at[i]], b2_v.at[0], gsem)

            @pl.when(valid(j))
            def _():
                cp3 = pltpu.async_copy(table_sh.at[idxd_v.at[j]], b1_v.at[1], gsem)
                cp4 = pltpu.async_copy(table_sh.at[idxs_v.at[j]], b2_v.at[1], gsem)
                cp1.wait()
                cp2.wait()
                base = (first + i) * GCH
                w1 = pltpu.async_copy(b1_v.at[0], g1_hbm.at[pl.ds(base, GCH)], wsem)
                w2 = pltpu.async_copy(b2_v.at[0], g2_hbm.at[pl.ds(base, GCH)], wsem)
                cp3.wait()
                cp4.wait()
                basej = (first + j) * GCH
                w3 = pltpu.async_copy(b1_v.at[1], g1_hbm.at[pl.ds(basej, GCH)], wsem)
                w4 = pltpu.async_copy(b2_v.at[1], g2_hbm.at[pl.ds(basej, GCH)], wsem)
                w1.wait()
                w2.wait()
                w3.wait()
                w4.wait()

            @pl.when(jnp.logical_not(valid(j)))
            def _():
                cp1.wait()
                cp2.wait()
                base = (first + i) * GCH
                w1 = pltpu.async_copy(b1_v.at[0], g1_hbm.at[pl.ds(base, GCH)], wsem)
                w2 = pltpu.async_copy(b2_v.at[0], g2_hbm.at[pl.ds(base, GCH)], wsem)
                w1.wait()
                w2.wait()
        return 0

    lax.fori_loop(0, GC // 2, pair, 0)


_sc_gather = functools.partial(
    pl.kernel,
    out_type=(jax.ShapeDtypeStruct((EHALF, D), jnp.float32),
              jax.ShapeDtypeStruct((EHALF, D), jnp.float32)),
    mesh=plsc.VectorSubcoreMesh(core_axis_name="c", subcore_axis_name="s",
                                num_cores=NC, num_subcores=NS),
    scratch_types=(pltpu.VMEM_SHARED((N, D), jnp.float32),
                   pltpu.VMEM((GC, GCH), jnp.int32),
                   pltpu.VMEM((GC, GCH), jnp.int32),
                   pltpu.VMEM((2, GCH, D), jnp.float32),
                   pltpu.VMEM((2, GCH, D), jnp.float32),
                   pltpu.SemaphoreType.DMA,
                   pltpu.SemaphoreType.DMA),
)(_gather_body)


ROWS_PER_TILE = 1000


def _scatter_body(msg_hbm, dstp_hbm, zeros_hbm, agg_hbm,
                  accum_sh, idx_v, mbuf_v, lsem, asem):
    c = lax.axis_index("c")
    s = lax.axis_index("s")
    first = s * SPER

    @pl.when(s < N // ROWS_PER_TILE)
    def _():
        pltpu.sync_copy(zeros_hbm,
                        accum_sh.at[pl.ds(s * ROWS_PER_TILE, ROWS_PER_TILE)])

    pltpu.sync_copy(dstp_hbm.at[pl.ds(first, SPER)], idx_v)
    plsc.subcore_barrier()

    def valid(i):
        return jnp.logical_and(i < SPER, first + i < NCHUNK)

    def pair(i2, _):
        i = i2 * 2
        j = i + 1

        @pl.when(valid(i))
        def _():
            l1 = pltpu.async_copy(msg_hbm.at[c, pl.ds((first + i) * C, C)],
                                  mbuf_v.at[0], lsem)

            @pl.when(valid(j))
            def _():
                l2 = pltpu.async_copy(msg_hbm.at[c, pl.ds((first + j) * C, C)],
                                      mbuf_v.at[1], lsem)
                l1.wait()
                a1 = pltpu.async_copy(mbuf_v.at[0], accum_sh.at[idx_v.at[i]],
                                      asem, add=True)
                l2.wait()
                a2 = pltpu.async_copy(mbuf_v.at[1], accum_sh.at[idx_v.at[j]],
                                      asem, add=True)
                a1.wait()
                a2.wait()

            @pl.when(jnp.logical_not(valid(j)))
            def _():
                l1.wait()
                a1 = pltpu.async_copy(mbuf_v.at[0], accum_sh.at[idx_v.at[i]],
                                      asem, add=True)
                a1.wait()
        return 0

    lax.fori_loop(0, SPER // 2, pair, 0)
    plsc.subcore_barrier()

    @pl.when(s < N // ROWS_PER_TILE)
    def _():
        pltpu.sync_copy(accum_sh.at[pl.ds(s * ROWS_PER_TILE, ROWS_PER_TILE)],
                        agg_hbm.at[c, pl.ds(s * ROWS_PER_TILE, ROWS_PER_TILE)])


_sc_scatter = functools.partial(
    pl.kernel,
    out_type=jax.ShapeDtypeStruct((NC, N, D), jnp.float32),
    mesh=plsc.VectorSubcoreMesh(core_axis_name="c", subcore_axis_name="s",
                                num_cores=NC, num_subcores=NS),
    scratch_types=(pltpu.VMEM_SHARED((N, D), jnp.float32),
                   pltpu.VMEM((SPER, C), jnp.int32),
                   pltpu.VMEM((2, C, D), jnp.float32),
                   pltpu.SemaphoreType.DMA,
                   pltpu.SemaphoreType.DMA),
)(_scatter_body)


EB = 8000


def _edge_mlp_body(g1_ref, g2_ref, wp_ref, wq_ref, b1_ref, w2_ref, b2_ref,
                   out_ref):
    bf = jnp.bfloat16
    pre = (jnp.dot(g1_ref[...].astype(bf), wp_ref[...],
                   preferred_element_type=jnp.float32)
           + jnp.dot(g2_ref[...].astype(bf), wq_ref[...],
                     preferred_element_type=jnp.float32)
           + b1_ref[...])
    h1 = _elu(pre)
    msg = _elu(jnp.dot(h1.astype(bf), w2_ref[...],
                       preferred_element_type=jnp.float32)
               + b2_ref[...])
    out_ref[0] = msg[:, :D]
    out_ref[1] = msg[:, D:]


def _tc_edge(g1, g2, wp, wq, b1, w2, b2):
    return pl.pallas_call(
        _edge_mlp_body,
        grid=(EHALF // EB,),
        in_specs=[
            pl.BlockSpec((EB, D), lambda i: (i, 0)),
            pl.BlockSpec((EB, D), lambda i: (i, 0)),
            pl.BlockSpec((D, 2 * D), lambda i: (0, 0)),
            pl.BlockSpec((D, 2 * D), lambda i: (0, 0)),
            pl.BlockSpec((1, 2 * D), lambda i: (0, 0)),
            pl.BlockSpec((2 * D, 2 * D), lambda i: (0, 0)),
            pl.BlockSpec((1, 2 * D), lambda i: (0, 0)),
        ],
        out_specs=pl.BlockSpec((NC, EB, D), lambda i: (0, i, 0)),
        out_shape=jax.ShapeDtypeStruct((NC, EHALF, D), jnp.float32),
    )(g1, g2, wp, wq, b1, w2, b2)


FRB = 400


def _final_body(agg_ref, aggb_ref, m2w1_ref, m2b1_ref, m2w2_ref, m2b2_ref,
                mow1_ref, mob1_ref, mow2_ref, mob2_ref,
                wi1_ref, bi1_ref, w_ref, b_ref, out_ref):
    bf = jnp.bfloat16
    x = jnp.concatenate([agg_ref[0] + aggb_ref[0],
                         agg_ref[1] + aggb_ref[1]], axis=1)
    u = _elu(jnp.dot(x.astype(bf), m2w1_ref[...],
                     preferred_element_type=jnp.float32) + m2b1_ref[...])
    u = _elu(jnp.dot(u.astype(bf), m2w2_ref[...],
                     preferred_element_type=jnp.float32) + m2b2_ref[...])
    o = _elu(jnp.dot(u.astype(bf), mow1_ref[...],
                     preferred_element_type=jnp.float32) + mob1_ref[...])
    o = _elu(jnp.dot(o.astype(bf), mow2_ref[...],
                     preferred_element_type=jnp.float32) + mob2_ref[...])
    o = o[:, D:]
    h = jnp.maximum(
        jnp.dot(o.astype(bf), wi1_ref[...], preferred_element_type=jnp.float32)
        + bi1_ref[...], 0.0)
    z = (jnp.dot(h.astype(bf), w_ref[...], preferred_element_type=jnp.float32)
         + b_ref[...])
    out_ref[...] = 0.5 + 0.5 * jnp.tanh(0.5 * z)


def _tc_final(agga, aggb, m2W1, m2b1, m2W2, m2b2, moW1, mob1, moW2, mob2,
              Wi1, bi1, Wi2, bi2):
    full = lambda r, c: pl.BlockSpec((r, c), lambda i: (0, 0))
    return pl.pallas_call(
        _final_body,
        grid=(N // FRB,),
        in_specs=[
            pl.BlockSpec((NC, FRB, D), lambda i: (0, i, 0)),
            pl.BlockSpec((NC, FRB, D), lambda i: (0, i, 0)),
            full(2 * D, 2 * D), full(1, 2 * D),
            full(2 * D, 2 * D), full(1, 2 * D),
            full(2 * D, 2 * D), full(1, 2 * D),
            full(2 * D, 2 * D), full(1, 2 * D),
            full(D, 3 * D), full(1, 3 * D),
            full(3 * D, NUM_API), full(1, NUM_API),
        ],
        out_specs=pl.BlockSpec((FRB, NUM_API), lambda i: (i, 0)),
        out_shape=jax.ShapeDtypeStruct((N, NUM_API), jnp.float32),
    )(agga, aggb, m2W1, m2b1.reshape(1, -1), m2W2, m2b2.reshape(1, -1),
      moW1, mob1.reshape(1, -1), moW2, mob2.reshape(1, -1),
      Wi1, bi1.reshape(1, -1), Wi2, bi2.reshape(1, -1))


def kernel(api_embeds, adjacency_matrix, edge_index,
           Wr1, br1, Wr2, br2,
           m1W1, m1b1, m1W2, m1b2,
           m2W1, m2b1, m2W2, m2b2,
           moW1, mob1, moW2, mob2,
           Wi1, bi1, Wi2, bi2):
    del adjacency_matrix, Wr1, br1, Wr2, br2
    src = edge_index[0]
    dst = edge_index[1]
    pad = ((0, NPAD - NCHUNK), (0, 0))
    gpad = ((0, GNPAD - GNCHUNK), (0, 0))
    dstp = [jnp.pad(dst[h * EHALF:(h + 1) * EHALF].reshape(NCHUNK, C), pad)
            for h in range(2)]
    srcp = [jnp.pad(src[h * EHALF:(h + 1) * EHALF].reshape(NCHUNK, C), pad)
            for h in range(2)]
    dstg = [jnp.pad(dst[h * EHALF:(h + 1) * EHALF].reshape(GNCHUNK, GCH), gpad)
            for h in range(2)]
    srcg = [jnp.pad(src[h * EHALF:(h + 1) * EHALF].reshape(GNCHUNK, GCH), gpad)
            for h in range(2)]

    bf = jnp.bfloat16
    WP = (m1W1[:D] + m1W1[D:2 * D]).astype(bf)
    WQ = (m1W1[2 * D:3 * D] + m1W1[3 * D:]).astype(bf)
    b1r = m1b1.reshape(1, -1)
    W2 = m1W2.astype(bf)
    b2r = m1b2.reshape(1, -1)
    zeros = jnp.zeros((ROWS_PER_TILE, D), jnp.float32)

    g1a, g2a = _sc_gather(api_embeds, dstg[0], srcg[0])
    g1b, g2b = _sc_gather(api_embeds, dstg[1], srcg[1])
    msga = _tc_edge(g1a, g2a, WP, WQ, b1r, W2, b2r)
    agga = _sc_scatter(msga, dstp[0], zeros)
    msgb = _tc_edge(g1b, g2b, WP, WQ, b1r, W2, b2r)
    aggb = _sc_scatter(msgb, dstp[1], zeros)
    return _tc_final(agga, aggb, m2W1.astype(bf), m2b1, m2W2.astype(bf), m2b2,
                     moW1.astype(bf), mob1, moW2.astype(bf), mob2,
                     Wi1.astype(bf), bi1, Wi2.astype(bf), bi2)
